# Initial kernel scaffold; baseline (speedup 1.0000x reference)
import jax
import jax.numpy as jnp
from jax.experimental import pallas as pl


def kernel(embeddings, edge_index, edge_type, W_fc, b_fc, bases1, comp1, root1, bias1, bases2, comp2, root2, bias2):
    raise NotImplementedError("write your pallas kernel here")


# speedup vs baseline: 3.5712x; 3.5712x over previous
import functools
import numpy as np
import jax
import jax.numpy as jnp
from jax import lax
from jax.experimental import pallas as pl
from jax.experimental.pallas import tpu as pltpu
from jax.experimental.pallas import tpu_sc as plsc

N = 10000
E = 160000
R = 8
D_EMB = 768
D_IN = 256
H = 16

NC, NS, LANES = 2, 16, 16
NW = NC * NS
CH = 128
EPW = 5120
NCHUNK = EPW // CH
EP = EPW * NW
TCHUNKS = EP // CH
NKEY = 81920
NPA = 10240

_AGG_TPW = NPA // NS
_KEY_TPW = NKEY // NS


def _deg_body(edges_hbm, zeros1_hbm, deg_out, edg_b, kidx_b, ones_b, zbuf1,
              deg_sp):
    sid = lax.axis_index("s")
    ones_v = jnp.ones((LANES,), jnp.float32)

    pltpu.sync_copy(zeros1_hbm, zbuf1)
    dslot0 = sid * _KEY_TPW
    pltpu.sync_copy(zbuf1, deg_sp.at[pl.ds(dslot0, _KEY_TPW)])
    for g in range(CH // LANES):
        ones_b[pl.ds(g * LANES, LANES)] = ones_v
    plsc.subcore_barrier()

    nchunk = TCHUNKS // NS

    def _chunk(j, _):
        gchunk = sid * nchunk + j
        pltpu.sync_copy(edges_hbm.at[gchunk], edg_b)
        for g in range(CH // LANES):
            off = pl.multiple_of(g * LANES, LANES)
            d = edg_b[1, pl.ds(off, LANES)]
            t = edg_b[2, pl.ds(off, LANES)]
            kidx_b[0, pl.ds(off, LANES)] = d * R + t
        pltpu.sync_copy(ones_b, deg_sp.at[kidx_b.at[0]], add=True)
        return 0
    lax.fori_loop(0, nchunk, _chunk, 0)

    plsc.subcore_barrier()
    pltpu.sync_copy(deg_sp.at[pl.ds(dslot0, _KEY_TPW)], zbuf1)
    pltpu.sync_copy(zbuf1, deg_out.at[pl.ds(dslot0, _KEY_TPW)])


def _make_deg_pass():
    mesh = plsc.VectorSubcoreMesh(core_axis_name="c", subcore_axis_name="s",
                                  num_cores=1, num_subcores=NS)
    return pl.kernel(
        _deg_body,
        out_type=jax.ShapeDtypeStruct((NKEY,), jnp.float32),
        mesh=mesh,
        scratch_types=[
            pltpu.VMEM((3, CH), jnp.int32),
            pltpu.VMEM((1, CH), jnp.int32),
            pltpu.VMEM((CH,), jnp.float32),
            pltpu.VMEM((_KEY_TPW,), jnp.float32),
            pltpu.VMEM_SHARED((NKEY,), jnp.float32),
        ],
        compiler_params=pltpu.CompilerParams(needs_layout_passes=False,
                                             use_tc_tiling_on_sc=False),
    )


def _edge_body(edges_hbm, z_hbm, recip_hbm, zeros_hbm, agg_out,
               edg_b, kidx_b, kpad_b, recip_b, rows_b, msg_b, zbuf, agg_sp):
    cid = lax.axis_index("c")
    sid = lax.axis_index("s")
    wid = sid * NC + cid
    lane = lax.iota(jnp.int32, LANES)

    pltpu.sync_copy(recip_hbm, recip_b)
    pltpu.sync_copy(zeros_hbm, zbuf)
    arow0 = sid * _AGG_TPW
    pltpu.sync_copy(zbuf, agg_sp.at[pl.ds(arow0, _AGG_TPW)])
    plsc.subcore_barrier()

    def _chunk(j, _):
        gchunk = wid * NCHUNK + j
        pltpu.sync_copy(edges_hbm.at[gchunk], edg_b)
        for g in range(CH // LANES):
            off = pl.multiple_of(g * LANES, LANES)
            d = edg_b[1, pl.ds(off, LANES)]
            t = edg_b[2, pl.ds(off, LANES)]
            k = d * R + t
            kidx_b[0, pl.ds(off, LANES)] = k
            kpad_b[pl.ds(LANES + off, LANES)] = k
        pltpu.sync_copy(z_hbm.at[edg_b.at[0]], rows_b)
        for l in range(CH):
            kspl = plsc.load_gather(
                kpad_b, [jnp.full((LANES,), LANES + l, jnp.int32)])
            nrm = plsc.load_gather(recip_b, [kspl])
            t_spl = lax.rem(kspl, jnp.full((LANES,), R, jnp.int32))
            y = plsc.load_gather(rows_b.at[l], [t_spl * H + lane])
            msg_b[l, :] = y * nrm
        for g in range(CH // LANES):
            off = pl.multiple_of(g * LANES, LANES)
            k = kidx_b[0, pl.ds(off, LANES)]
            kidx_b[0, pl.ds(off, LANES)] = lax.div(
                k, jnp.full((LANES,), R, jnp.int32))
        pltpu.sync_copy(msg_b, agg_sp.at[kidx_b.at[0]], add=True)
        return 0
    lax.fori_loop(0, NCHUNK, _chunk, 0)

    plsc.subcore_barrier()
    pltpu.sync_copy(agg_sp.at[pl.ds(arow0, _AGG_TPW)], zbuf)
    pltpu.sync_copy(zbuf, agg_out.at[cid, pl.ds(arow0, _AGG_TPW)])


def _make_edge_pass():
    mesh = plsc.VectorSubcoreMesh(core_axis_name="c", subcore_axis_name="s",
                                  num_cores=NC, num_subcores=NS)
    return pl.kernel(
        _edge_body,
        out_type=jax.ShapeDtypeStruct((NC, NPA, LANES), jnp.float32),
        mesh=mesh,
        scratch_types=[
            pltpu.VMEM((3, CH), jnp.int32),
            pltpu.VMEM((1, CH), jnp.int32),
            pltpu.VMEM((CH + LANES,), jnp.int32),
            pltpu.VMEM((NKEY,), jnp.float32),
            pltpu.VMEM((CH, R * H), jnp.float32),
            pltpu.VMEM((CH, LANES), jnp.float32),
            pltpu.VMEM((_AGG_TPW, LANES), jnp.float32),
            pltpu.VMEM_SHARED((NPA, LANES), jnp.float32),
        ],
        compiler_params=pltpu.CompilerParams(needs_layout_passes=False,
                                             use_tc_tiling_on_sc=False),
    )


_RB = 1000
_DB = NKEY // 128 // 10


def _tc1_body(emb, wfc, bfc, wcat, root, bias, deg, z_out, rp_out, recip_out):
    h = jnp.dot(emb[...], wfc[...], preferred_element_type=jnp.float32) + bfc[...]
    z_out[...] = jnp.dot(h, wcat[...], preferred_element_type=jnp.float32)
    rp_out[...] = jnp.dot(h, root[...], preferred_element_type=jnp.float32) + bias[...]
    recip_out[...] = 1.0 / jnp.maximum(deg[...], 1.0)


def _tc1(emb, wfc, bfc, wcat, root, bias, deg):
    f32 = jnp.float32
    full = lambda shape: pl.BlockSpec(shape, lambda i: (0, 0))
    return pl.pallas_call(
        _tc1_body,
        grid=(N // _RB,),
        in_specs=[
            pl.BlockSpec((_RB, D_EMB), lambda i: (i, 0)),
            full((D_EMB, D_IN)), full((1, D_IN)),
            full((D_IN, R * H)), full((D_IN, H)), full((1, H)),
            pl.BlockSpec((_DB, 128), lambda i: (i, 0)),
        ],
        out_specs=[pl.BlockSpec((_RB, R * H), lambda i: (i, 0)),
                   pl.BlockSpec((_RB, H), lambda i: (i, 0)),
                   pl.BlockSpec((_DB, 128), lambda i: (i, 0))],
        out_shape=[jax.ShapeDtypeStruct((N, R * H), f32),
                   jax.ShapeDtypeStruct((N, H), f32),
                   jax.ShapeDtypeStruct((NKEY // 128, 128), f32)],
    )(emb, wfc, bfc, wcat, root, bias, deg)


def _tc2_body(a0, a1, rp1, wcat2, root2, bias2, z2_out, rp2_out):
    x1 = jnp.maximum(a0[...] + a1[...] + rp1[...], 0.0)
    z2_out[...] = jnp.dot(x1, wcat2[...], preferred_element_type=jnp.float32)
    rp2_out[...] = jnp.dot(x1, root2[...], preferred_element_type=jnp.float32) + bias2[...]


def _tc2(a0, a1, rp1, wcat2, root2, bias2):
    f32 = jnp.float32
    full = lambda shape: pl.BlockSpec(shape, lambda i: (0, 0))
    blk = lambda w: pl.BlockSpec((_RB, w), lambda i: (i, 0))
    return pl.pallas_call(
        _tc2_body,
        grid=(N // _RB,),
        in_specs=[blk(H), blk(H), blk(H),
                  full((H, R * H)), full((H, H)), full((1, H))],
        out_specs=[blk(R * H), blk(H)],
        out_shape=[jax.ShapeDtypeStruct((N, R * H), f32),
                   jax.ShapeDtypeStruct((N, H), f32)],
    )(a0, a1, rp1, wcat2, root2, bias2)


def _tc3_body(a0, a1, rp2, out):
    out[...] = a0[...] + a1[...] + rp2[...]


def _tc3(a0, a1, rp2):
    blk = lambda w: pl.BlockSpec((_RB, w), lambda i: (i, 0))
    return pl.pallas_call(
        _tc3_body,
        grid=(N // _RB,),
        in_specs=[blk(H), blk(H), blk(H)],
        out_specs=blk(H),
        out_shape=jax.ShapeDtypeStruct((N, H), jnp.float32),
    )(a0, a1, rp2)


def kernel(embeddings, edge_index, edge_type, W_fc, b_fc,
           bases1, comp1, root1, bias1, bases2, comp2, root2, bias2):
    f32 = jnp.float32
    src = edge_index[0].astype(jnp.int32)
    dst = edge_index[1].astype(jnp.int32)
    et = edge_type.astype(jnp.int32)
    npad = EP - E
    src_p = jnp.concatenate([src, jnp.zeros((npad,), jnp.int32)])
    dst_p = jnp.concatenate([dst, jnp.full((npad,), N, jnp.int32)])
    et_p = jnp.concatenate([et, jnp.zeros((npad,), jnp.int32)])
    edges3 = jnp.stack([src_p.reshape(TCHUNKS, CH),
                        dst_p.reshape(TCHUNKS, CH),
                        et_p.reshape(TCHUNKS, CH)], axis=1)

    wcat1 = jnp.einsum('rb,bio->rio', comp1, bases1).transpose(1, 0, 2).reshape(D_IN, R * H)
    wcat2 = jnp.einsum('rb,bio->rio', comp2, bases2).transpose(1, 0, 2).reshape(H, R * H)
    zeros_hbm = jnp.zeros((_AGG_TPW, LANES), f32)
    zeros1_hbm = jnp.zeros((_KEY_TPW,), f32)

    deg = _make_deg_pass()(edges3, zeros1_hbm)

    z1, rp1, recip = _tc1(embeddings, W_fc, b_fc.reshape(1, D_IN), wcat1,
                          root1, bias1.reshape(1, H), deg.reshape(NKEY // 128, 128))
    recip_flat = recip.reshape(NKEY)

    edge_pass = _make_edge_pass()
    agg1 = edge_pass(edges3, z1, recip_flat, zeros_hbm)
    a10 = agg1[0, :N]
    a11 = agg1[1, :N]

    z2, rp2 = _tc2(a10, a11, rp1, wcat2, root2, bias2.reshape(1, H))

    agg2 = edge_pass(edges3, z2, recip_flat, zeros_hbm)

    return _tc3(agg2[0, :N], agg2[1, :N], rp2)


# speedup vs baseline: 3.7382x; 1.0468x over previous
import functools
import numpy as np
import jax
import jax.numpy as jnp
from jax import lax
from jax.experimental import pallas as pl
from jax.experimental.pallas import tpu as pltpu
from jax.experimental.pallas import tpu_sc as plsc

N = 10000
E = 160000
R = 8
D_EMB = 768
D_IN = 256
H = 16

NC, NS, LANES = 2, 16, 16
NW = NC * NS
CH = 128
EPW = 5120
NCHUNK = EPW // CH
EP = EPW * NW
TCHUNKS = EP // CH
NKEY = 81920
NPA = 10240

_AGG_TPW = NPA // NS
_KEY_TPW = NKEY // NS


def _deg_body(edges_hbm, zeros1_hbm, deg_out, edg_b, kidx_b, ones_b,
              se0, se1, ss0, ss1, deg_sp):
    sid = lax.axis_index("s")
    ones_v = jnp.ones((LANES,), jnp.float32)
    nchunk = TCHUNKS // NS
    se = (se0, se1)
    ss = (ss0, ss1)

    dslot0 = sid * _KEY_TPW
    pltpu.sync_copy(zeros1_hbm, deg_sp.at[pl.ds(dslot0, _KEY_TPW)])
    for g in range(CH // LANES):
        ones_b[pl.ds(g * LANES, LANES)] = ones_v
    plsc.subcore_barrier()

    def start_edges(j, b):
        pltpu.async_copy(edges_hbm.at[sid * nchunk + j], edg_b.at[b], se[b])

    def wait_edges(b):
        pltpu.make_async_copy(edges_hbm.at[0], edg_b.at[b], se[b]).wait()

    def wait_scatter(b):
        pltpu.make_async_copy(ones_b, deg_sp.at[kidx_b.at[b]], ss[b]).wait()

    start_edges(0, 0)

    def _chunk(j, _):
        def it(b):
            ob = 1 - b
            wait_edges(b)
            @pl.when(j + 1 < nchunk)
            def _():
                start_edges(j + 1, ob)
            @pl.when(j >= 2)
            def _():
                wait_scatter(b)
            for g in range(CH // LANES):
                off = pl.multiple_of(g * LANES, LANES)
                d = edg_b[b, 1, pl.ds(off, LANES)]
                t = edg_b[b, 2, pl.ds(off, LANES)]
                kidx_b[b, pl.ds(off, LANES)] = d * R + t
            pltpu.async_copy(ones_b, deg_sp.at[kidx_b.at[b]], ss[b], add=True)
        @pl.when(lax.rem(j, 2) == 0)
        def _():
            it(0)
        @pl.when(lax.rem(j, 2) == 1)
        def _():
            it(1)
        return 0
    lax.fori_loop(0, nchunk, _chunk, 0)
    wait_scatter(0)
    wait_scatter(1)

    plsc.subcore_barrier()
    pltpu.sync_copy(deg_sp.at[pl.ds(dslot0, _KEY_TPW)],
                    deg_out.at[pl.ds(dslot0, _KEY_TPW)])


def _make_deg_pass():
    mesh = plsc.VectorSubcoreMesh(core_axis_name="c", subcore_axis_name="s",
                                  num_cores=1, num_subcores=NS)
    return pl.kernel(
        _deg_body,
        out_type=jax.ShapeDtypeStruct((NKEY,), jnp.float32),
        mesh=mesh,
        scratch_types=[
            pltpu.VMEM((2, 3, CH), jnp.int32),
            pltpu.VMEM((2, CH), jnp.int32),
            pltpu.VMEM((CH,), jnp.float32),
            pltpu.SemaphoreType.DMA, pltpu.SemaphoreType.DMA,
            pltpu.SemaphoreType.DMA, pltpu.SemaphoreType.DMA,
            pltpu.VMEM_SHARED((NKEY,), jnp.float32),
        ],
        compiler_params=pltpu.CompilerParams(needs_layout_passes=False,
                                             use_tc_tiling_on_sc=False),
    )


def _edge_body(edges_hbm, z_hbm, recip_hbm, zeros_hbm, agg_out,
               edg_b, kidx_b, kpad_b, recip_b, rows_b, msg_b,
               se0, se1, sg0, sg1, ss0, ss1, agg_sp):
    cid = lax.axis_index("c")
    sid = lax.axis_index("s")
    wid = sid * NC + cid
    lane = lax.iota(jnp.int32, LANES)
    se, sg, ss = (se0, se1), (sg0, sg1), (ss0, ss1)

    pltpu.sync_copy(recip_hbm, recip_b)
    arow0 = sid * _AGG_TPW
    pltpu.sync_copy(zeros_hbm, agg_sp.at[pl.ds(arow0, _AGG_TPW)])
    plsc.subcore_barrier()

    def start_edges(j, b):
        pltpu.async_copy(edges_hbm.at[wid * NCHUNK + j], edg_b.at[b], se[b])

    def wait_edges(b):
        pltpu.make_async_copy(edges_hbm.at[0], edg_b.at[b], se[b]).wait()

    def compute_keys(b):
        for g in range(CH // LANES):
            off = pl.multiple_of(g * LANES, LANES)
            d = edg_b[b, 1, pl.ds(off, LANES)]
            t = edg_b[b, 2, pl.ds(off, LANES)]
            kidx_b[b, pl.ds(off, LANES)] = d
            kpad_b[b, pl.ds(LANES + off, LANES)] = d * R + t

    def start_gather(b):
        pltpu.async_copy(z_hbm.at[edg_b.at[b, 0]], rows_b.at[b], sg[b])

    def wait_gather(b):
        pltpu.make_async_copy(z_hbm.at[edg_b.at[b, 0]], rows_b.at[b],
                              sg[b]).wait()

    def compute_msgs(b):
        for l in range(CH):
            kspl = plsc.load_gather(
                kpad_b.at[b], [jnp.full((LANES,), LANES + l, jnp.int32)])
            nrm = plsc.load_gather(recip_b, [kspl])
            t_spl = lax.rem(kspl, jnp.full((LANES,), R, jnp.int32))
            y = plsc.load_gather(rows_b.at[b, l], [t_spl * H + lane])
            msg_b[b, l, :] = y * nrm

    def start_scatter(b):
        pltpu.async_copy(msg_b.at[b], agg_sp.at[kidx_b.at[b]], ss[b], add=True)

    def wait_scatter(b):
        pltpu.make_async_copy(msg_b.at[b], agg_sp.at[kidx_b.at[b]],
                              ss[b]).wait()

    start_edges(jnp.int32(0), 0)
    wait_edges(0)
    compute_keys(0)
    start_gather(0)
    start_edges(jnp.int32(1), 1)

    def _chunk(j, _):
        def it(b):
            ob = 1 - b
            wait_gather(b)
            compute_msgs(b)
            start_scatter(b)
            @pl.when(j + 1 < NCHUNK)
            def _():
                wait_edges(ob)
                @pl.when(j >= 1)
                def _():
                    wait_scatter(ob)
                compute_keys(ob)
                start_gather(ob)
                @pl.when(j + 2 < NCHUNK)
                def _():
                    start_edges(j + 2, b)
        @pl.when(lax.rem(j, 2) == 0)
        def _():
            it(0)
        @pl.when(lax.rem(j, 2) == 1)
        def _():
            it(1)
        return 0
    lax.fori_loop(0, NCHUNK, _chunk, 0)
    wait_scatter(0)
    wait_scatter(1)

    plsc.subcore_barrier()
    pltpu.sync_copy(agg_sp.at[pl.ds(arow0, _AGG_TPW)],
                    agg_out.at[cid, pl.ds(arow0, _AGG_TPW)])


def _make_edge_pass():
    mesh = plsc.VectorSubcoreMesh(core_axis_name="c", subcore_axis_name="s",
                                  num_cores=NC, num_subcores=NS)
    return pl.kernel(
        _edge_body,
        out_type=jax.ShapeDtypeStruct((NC, NPA, LANES), jnp.float32),
        mesh=mesh,
        scratch_types=[
            pltpu.VMEM((2, 3, CH), jnp.int32),
            pltpu.VMEM((2, CH), jnp.int32),
            pltpu.VMEM((2, CH + LANES), jnp.int32),
            pltpu.VMEM((NKEY,), jnp.float32),
            pltpu.VMEM((2, CH, R * H), jnp.float32),
            pltpu.VMEM((2, CH, LANES), jnp.float32),
            pltpu.SemaphoreType.DMA, pltpu.SemaphoreType.DMA,
            pltpu.SemaphoreType.DMA, pltpu.SemaphoreType.DMA,
            pltpu.SemaphoreType.DMA, pltpu.SemaphoreType.DMA,
            pltpu.VMEM_SHARED((NPA, LANES), jnp.float32),
        ],
        compiler_params=pltpu.CompilerParams(needs_layout_passes=False,
                                             use_tc_tiling_on_sc=False),
    )


_RB = 1000
_DB = NKEY // 128 // 10


def _tc1_body(emb, wfc, bfc, wcat, root, bias, deg, z_out, rp_out, recip_out):
    h = jnp.dot(emb[...], wfc[...], preferred_element_type=jnp.float32) + bfc[...]
    z_out[...] = jnp.dot(h, wcat[...], preferred_element_type=jnp.float32)
    rp_out[...] = jnp.dot(h, root[...], preferred_element_type=jnp.float32) + bias[...]
    recip_out[...] = 1.0 / jnp.maximum(deg[...], 1.0)


def _tc1(emb, wfc, bfc, wcat, root, bias, deg):
    f32 = jnp.float32
    full = lambda shape: pl.BlockSpec(shape, lambda i: (0, 0))
    return pl.pallas_call(
        _tc1_body,
        grid=(N // _RB,),
        in_specs=[
            pl.BlockSpec((_RB, D_EMB), lambda i: (i, 0)),
            full((D_EMB, D_IN)), full((1, D_IN)),
            full((D_IN, R * H)), full((D_IN, H)), full((1, H)),
            pl.BlockSpec((_DB, 128), lambda i: (i, 0)),
        ],
        out_specs=[pl.BlockSpec((_RB, R * H), lambda i: (i, 0)),
                   pl.BlockSpec((_RB, H), lambda i: (i, 0)),
                   pl.BlockSpec((_DB, 128), lambda i: (i, 0))],
        out_shape=[jax.ShapeDtypeStruct((N, R * H), f32),
                   jax.ShapeDtypeStruct((N, H), f32),
                   jax.ShapeDtypeStruct((NKEY // 128, 128), f32)],
    )(emb, wfc, bfc, wcat, root, bias, deg)


def _tc2_body(a0, a1, rp1, wcat2, root2, bias2, z2_out, rp2_out):
    x1 = jnp.maximum(a0[...] + a1[...] + rp1[...], 0.0)
    z2_out[...] = jnp.dot(x1, wcat2[...], preferred_element_type=jnp.float32)
    rp2_out[...] = jnp.dot(x1, root2[...], preferred_element_type=jnp.float32) + bias2[...]


def _tc2(a0, a1, rp1, wcat2, root2, bias2):
    f32 = jnp.float32
    full = lambda shape: pl.BlockSpec(shape, lambda i: (0, 0))
    blk = lambda w: pl.BlockSpec((_RB, w), lambda i: (i, 0))
    return pl.pallas_call(
        _tc2_body,
        grid=(N // _RB,),
        in_specs=[blk(H), blk(H), blk(H),
                  full((H, R * H)), full((H, H)), full((1, H))],
        out_specs=[blk(R * H), blk(H)],
        out_shape=[jax.ShapeDtypeStruct((N, R * H), f32),
                   jax.ShapeDtypeStruct((N, H), f32)],
    )(a0, a1, rp1, wcat2, root2, bias2)


def _tc3_body(a0, a1, rp2, out):
    out[...] = a0[...] + a1[...] + rp2[...]


def _tc3(a0, a1, rp2):
    blk = lambda w: pl.BlockSpec((_RB, w), lambda i: (i, 0))
    return pl.pallas_call(
        _tc3_body,
        grid=(N // _RB,),
        in_specs=[blk(H), blk(H), blk(H)],
        out_specs=blk(H),
        out_shape=jax.ShapeDtypeStruct((N, H), jnp.float32),
    )(a0, a1, rp2)


def kernel(embeddings, edge_index, edge_type, W_fc, b_fc,
           bases1, comp1, root1, bias1, bases2, comp2, root2, bias2):
    f32 = jnp.float32
    src = edge_index[0].astype(jnp.int32)
    dst = edge_index[1].astype(jnp.int32)
    et = edge_type.astype(jnp.int32)
    npad = EP - E
    src_p = jnp.concatenate([src, jnp.zeros((npad,), jnp.int32)])
    dst_p = jnp.concatenate([dst, jnp.full((npad,), N, jnp.int32)])
    et_p = jnp.concatenate([et, jnp.zeros((npad,), jnp.int32)])
    edges3 = jnp.stack([src_p.reshape(TCHUNKS, CH),
                        dst_p.reshape(TCHUNKS, CH),
                        et_p.reshape(TCHUNKS, CH)], axis=1)

    wcat1 = jnp.einsum('rb,bio->rio', comp1, bases1).transpose(1, 0, 2).reshape(D_IN, R * H)
    wcat2 = jnp.einsum('rb,bio->rio', comp2, bases2).transpose(1, 0, 2).reshape(H, R * H)
    zeros_hbm = jnp.zeros((_AGG_TPW, LANES), f32)
    zeros1_hbm = jnp.zeros((_KEY_TPW,), f32)

    deg = _make_deg_pass()(edges3, zeros1_hbm)

    z1, rp1, recip = _tc1(embeddings, W_fc, b_fc.reshape(1, D_IN), wcat1,
                          root1, bias1.reshape(1, H), deg.reshape(NKEY // 128, 128))
    recip_flat = recip.reshape(NKEY)

    edge_pass = _make_edge_pass()
    agg1 = edge_pass(edges3, z1, recip_flat, zeros_hbm)
    a10 = agg1[0, :N]
    a11 = agg1[1, :N]

    z2, rp2 = _tc2(a10, a11, rp1, wcat2, root2, bias2.reshape(1, H))

    agg2 = edge_pass(edges3, z2, recip_flat, zeros_hbm)

    return _tc3(agg2[0, :N], agg2[1, :N], rp2)


# speedup vs baseline: 9.0776x; 2.4283x over previous
import functools
import numpy as np
import jax
import jax.numpy as jnp
from jax import lax
from jax.experimental import pallas as pl
from jax.experimental.pallas import tpu as pltpu
from jax.experimental.pallas import tpu_sc as plsc

N = 10000
E = 160000
R = 8
D_EMB = 768
D_IN = 256
H = 16

NC, NS, LANES = 2, 16, 16
NW = NC * NS
CH = 128
EPW = 5120
NCHUNK = EPW // CH
EP = EPW * NW
TCHUNKS = EP // CH
NKEY = 81920
NPA = 10240

_AGG_TPW = NPA // NS
_KEY_TPW = NKEY // NS


def _deg_body(edges_hbm, zeros1_hbm, deg_out, edg_b, kidx_b, ones_b,
              se0, se1, ss0, ss1, deg_sp):
    sid = lax.axis_index("s")
    ones_v = jnp.ones((LANES,), jnp.float32)
    nchunk = TCHUNKS // NS
    se = (se0, se1)
    ss = (ss0, ss1)

    dslot0 = sid * _KEY_TPW
    pltpu.sync_copy(zeros1_hbm, deg_sp.at[pl.ds(dslot0, _KEY_TPW)])
    for g in range(CH // LANES):
        ones_b[pl.ds(g * LANES, LANES)] = ones_v
    plsc.subcore_barrier()

    def start_edges(j, b):
        pltpu.async_copy(edges_hbm.at[sid * nchunk + j], edg_b.at[b], se[b])

    def wait_edges(b):
        pltpu.make_async_copy(edges_hbm.at[0], edg_b.at[b], se[b]).wait()

    def wait_scatter(b):
        pltpu.make_async_copy(ones_b, deg_sp.at[kidx_b.at[b]], ss[b]).wait()

    start_edges(0, 0)

    def _chunk(j, _):
        def it(b):
            ob = 1 - b
            wait_edges(b)
            @pl.when(j + 1 < nchunk)
            def _():
                start_edges(j + 1, ob)
            @pl.when(j >= 2)
            def _():
                wait_scatter(b)
            for g in range(CH // LANES):
                off = pl.multiple_of(g * LANES, LANES)
                d = edg_b[b, 1, pl.ds(off, LANES)]
                t = edg_b[b, 2, pl.ds(off, LANES)]
                kidx_b[b, pl.ds(off, LANES)] = d * R + t
            pltpu.async_copy(ones_b, deg_sp.at[kidx_b.at[b]], ss[b], add=True)
        @pl.when(lax.rem(j, 2) == 0)
        def _():
            it(0)
        @pl.when(lax.rem(j, 2) == 1)
        def _():
            it(1)
        return 0
    lax.fori_loop(0, nchunk, _chunk, 0)
    wait_scatter(0)
    wait_scatter(1)

    plsc.subcore_barrier()
    pltpu.sync_copy(deg_sp.at[pl.ds(dslot0, _KEY_TPW)],
                    deg_out.at[pl.ds(dslot0, _KEY_TPW)])


def _make_deg_pass():
    mesh = plsc.VectorSubcoreMesh(core_axis_name="c", subcore_axis_name="s",
                                  num_cores=1, num_subcores=NS)
    return pl.kernel(
        _deg_body,
        out_type=jax.ShapeDtypeStruct((NKEY,), jnp.float32),
        mesh=mesh,
        scratch_types=[
            pltpu.VMEM((2, 3, CH), jnp.int32),
            pltpu.VMEM((2, CH), jnp.int32),
            pltpu.VMEM((CH,), jnp.float32),
            pltpu.SemaphoreType.DMA, pltpu.SemaphoreType.DMA,
            pltpu.SemaphoreType.DMA, pltpu.SemaphoreType.DMA,
            pltpu.VMEM_SHARED((NKEY,), jnp.float32),
        ],
        compiler_params=pltpu.CompilerParams(needs_layout_passes=False,
                                             use_tc_tiling_on_sc=False),
    )


def _edge_body(edges_hbm, z_hbm, recip_hbm, zeros_hbm, agg_out,
               edg_b, kidx_b, gkey_b, npad_b, recip_b, rows_b,
               se0, se1, sg0, sg1, ss0, ss1, agg_sp):
    cid = lax.axis_index("c")
    sid = lax.axis_index("s")
    wid = sid * NC + cid
    se, sg, ss = (se0, se1), (sg0, sg1), (ss0, ss1)

    pltpu.sync_copy(recip_hbm, recip_b)
    arow0 = sid * _AGG_TPW
    pltpu.sync_copy(zeros_hbm, agg_sp.at[pl.ds(arow0, _AGG_TPW)])
    plsc.subcore_barrier()

    def start_edges(j, b):
        pltpu.async_copy(edges_hbm.at[wid * NCHUNK + j], edg_b.at[b], se[b])

    def wait_edges(b):
        pltpu.make_async_copy(edges_hbm.at[0], edg_b.at[b], se[b]).wait()

    def compute_keys(b):
        for g in range(CH // LANES):
            off = pl.multiple_of(g * LANES, LANES)
            s = edg_b[b, 0, pl.ds(off, LANES)]
            d = edg_b[b, 1, pl.ds(off, LANES)]
            t = edg_b[b, 2, pl.ds(off, LANES)]
            kidx_b[b, pl.ds(off, LANES)] = d
            gkey_b[b, pl.ds(off, LANES)] = s * R + t
            nrm = plsc.load_gather(recip_b, [d * R + t])
            npad_b[b, pl.ds(LANES + off, LANES)] = nrm

    def start_gather(b):
        pltpu.async_copy(z_hbm.at[gkey_b.at[b]], rows_b.at[b], sg[b])

    def wait_gather(b):
        pltpu.make_async_copy(z_hbm.at[gkey_b.at[b]], rows_b.at[b],
                              sg[b]).wait()

    def compute_msgs(b):
        for l in range(CH):
            nspl = plsc.load_gather(
                npad_b.at[b], [jnp.full((LANES,), LANES + l, jnp.int32)])
            rows_b[b, l, :] = rows_b[b, l, :] * nspl

    def start_scatter(b):
        pltpu.async_copy(rows_b.at[b], agg_sp.at[kidx_b.at[b]], ss[b],
                         add=True)

    def wait_scatter(b):
        pltpu.make_async_copy(rows_b.at[b], agg_sp.at[kidx_b.at[b]],
                              ss[b]).wait()

    start_edges(jnp.int32(0), 0)
    wait_edges(0)
    compute_keys(0)
    start_gather(0)
    start_edges(jnp.int32(1), 1)

    def _chunk(j, _):
        def it(b):
            ob = 1 - b
            wait_gather(b)
            compute_msgs(b)
            start_scatter(b)
            @pl.when(j + 1 < NCHUNK)
            def _():
                wait_edges(ob)
                @pl.when(j >= 1)
                def _():
                    wait_scatter(ob)
                compute_keys(ob)
                start_gather(ob)
                @pl.when(j + 2 < NCHUNK)
                def _():
                    start_edges(j + 2, b)
        @pl.when(lax.rem(j, 2) == 0)
        def _():
            it(0)
        @pl.when(lax.rem(j, 2) == 1)
        def _():
            it(1)
        return 0
    lax.fori_loop(0, NCHUNK, _chunk, 0)
    wait_scatter(0)
    wait_scatter(1)

    plsc.subcore_barrier()
    pltpu.sync_copy(agg_sp.at[pl.ds(arow0, _AGG_TPW)],
                    agg_out.at[cid, pl.ds(arow0, _AGG_TPW)])


def _make_edge_pass():
    mesh = plsc.VectorSubcoreMesh(core_axis_name="c", subcore_axis_name="s",
                                  num_cores=NC, num_subcores=NS)
    return pl.kernel(
        _edge_body,
        out_type=jax.ShapeDtypeStruct((NC, NPA, LANES), jnp.float32),
        mesh=mesh,
        scratch_types=[
            pltpu.VMEM((2, 3, CH), jnp.int32),
            pltpu.VMEM((2, CH), jnp.int32),
            pltpu.VMEM((2, CH), jnp.int32),
            pltpu.VMEM((2, CH + LANES), jnp.float32),
            pltpu.VMEM((NKEY,), jnp.float32),
            pltpu.VMEM((2, CH, LANES), jnp.float32),
            pltpu.SemaphoreType.DMA, pltpu.SemaphoreType.DMA,
            pltpu.SemaphoreType.DMA, pltpu.SemaphoreType.DMA,
            pltpu.SemaphoreType.DMA, pltpu.SemaphoreType.DMA,
            pltpu.VMEM_SHARED((NPA, LANES), jnp.float32),
        ],
        compiler_params=pltpu.CompilerParams(needs_layout_passes=False,
                                             use_tc_tiling_on_sc=False),
    )


_RB = 1000
_DB = NKEY // 128 // 10


def _tc1_body(emb, wfc, bfc, wcat, root, bias, deg, z_out, rp_out, recip_out):
    h = jnp.dot(emb[...], wfc[...], preferred_element_type=jnp.float32) + bfc[...]
    z_out[...] = jnp.dot(h, wcat[...], preferred_element_type=jnp.float32)
    rp_out[...] = jnp.dot(h, root[...], preferred_element_type=jnp.float32) + bias[...]
    recip_out[...] = 1.0 / jnp.maximum(deg[...], 1.0)


def _tc1(emb, wfc, bfc, wcat, root, bias, deg):
    f32 = jnp.float32
    full = lambda shape: pl.BlockSpec(shape, lambda i: (0, 0))
    return pl.pallas_call(
        _tc1_body,
        grid=(N // _RB,),
        in_specs=[
            pl.BlockSpec((_RB, D_EMB), lambda i: (i, 0)),
            full((D_EMB, D_IN)), full((1, D_IN)),
            full((D_IN, R * H)), full((D_IN, H)), full((1, H)),
            pl.BlockSpec((_DB, 128), lambda i: (i, 0)),
        ],
        out_specs=[pl.BlockSpec((_RB, R * H), lambda i: (i, 0)),
                   pl.BlockSpec((_RB, H), lambda i: (i, 0)),
                   pl.BlockSpec((_DB, 128), lambda i: (i, 0))],
        out_shape=[jax.ShapeDtypeStruct((N, R * H), f32),
                   jax.ShapeDtypeStruct((N, H), f32),
                   jax.ShapeDtypeStruct((NKEY // 128, 128), f32)],
    )(emb, wfc, bfc, wcat, root, bias, deg)


def _tc2_body(a0, a1, rp1, wcat2, root2, bias2, z2_out, rp2_out):
    x1 = jnp.maximum(a0[...] + a1[...] + rp1[...], 0.0)
    z2_out[...] = jnp.dot(x1, wcat2[...], preferred_element_type=jnp.float32)
    rp2_out[...] = jnp.dot(x1, root2[...], preferred_element_type=jnp.float32) + bias2[...]


def _tc2(a0, a1, rp1, wcat2, root2, bias2):
    f32 = jnp.float32
    full = lambda shape: pl.BlockSpec(shape, lambda i: (0, 0))
    blk = lambda w: pl.BlockSpec((_RB, w), lambda i: (i, 0))
    return pl.pallas_call(
        _tc2_body,
        grid=(N // _RB,),
        in_specs=[blk(H), blk(H), blk(H),
                  full((H, R * H)), full((H, H)), full((1, H))],
        out_specs=[blk(R * H), blk(H)],
        out_shape=[jax.ShapeDtypeStruct((N, R * H), f32),
                   jax.ShapeDtypeStruct((N, H), f32)],
    )(a0, a1, rp1, wcat2, root2, bias2)


def _tc3_body(a0, a1, rp2, out):
    out[...] = a0[...] + a1[...] + rp2[...]


def _tc3(a0, a1, rp2):
    blk = lambda w: pl.BlockSpec((_RB, w), lambda i: (i, 0))
    return pl.pallas_call(
        _tc3_body,
        grid=(N // _RB,),
        in_specs=[blk(H), blk(H), blk(H)],
        out_specs=blk(H),
        out_shape=jax.ShapeDtypeStruct((N, H), jnp.float32),
    )(a0, a1, rp2)


def kernel(embeddings, edge_index, edge_type, W_fc, b_fc,
           bases1, comp1, root1, bias1, bases2, comp2, root2, bias2):
    f32 = jnp.float32
    src = edge_index[0].astype(jnp.int32)
    dst = edge_index[1].astype(jnp.int32)
    et = edge_type.astype(jnp.int32)
    npad = EP - E
    src_p = jnp.concatenate([src, jnp.zeros((npad,), jnp.int32)])
    dst_p = jnp.concatenate([dst, jnp.full((npad,), N, jnp.int32)])
    et_p = jnp.concatenate([et, jnp.zeros((npad,), jnp.int32)])
    edges3 = jnp.stack([src_p.reshape(TCHUNKS, CH),
                        dst_p.reshape(TCHUNKS, CH),
                        et_p.reshape(TCHUNKS, CH)], axis=1)

    wcat1 = jnp.einsum('rb,bio->rio', comp1, bases1).transpose(1, 0, 2).reshape(D_IN, R * H)
    wcat2 = jnp.einsum('rb,bio->rio', comp2, bases2).transpose(1, 0, 2).reshape(H, R * H)
    zeros_hbm = jnp.zeros((_AGG_TPW, LANES), f32)
    zeros1_hbm = jnp.zeros((_KEY_TPW,), f32)

    deg = _make_deg_pass()(edges3, zeros1_hbm)

    z1, rp1, recip = _tc1(embeddings, W_fc, b_fc.reshape(1, D_IN), wcat1,
                          root1, bias1.reshape(1, H), deg.reshape(NKEY // 128, 128))
    recip_flat = recip.reshape(NKEY)

    edge_pass = _make_edge_pass()
    agg1 = edge_pass(edges3, z1.reshape(N * R, LANES), recip_flat, zeros_hbm)
    a10 = agg1[0, :N]
    a11 = agg1[1, :N]

    z2, rp2 = _tc2(a10, a11, rp1, wcat2, root2, bias2.reshape(1, H))

    agg2 = edge_pass(edges3, z2.reshape(N * R, LANES), recip_flat, zeros_hbm)

    return _tc3(agg2[0, :N], agg2[1, :N], rp2)


# speedup vs baseline: 9.3242x; 1.0272x over previous
import functools
import numpy as np
import jax
import jax.numpy as jnp
from jax import lax
from jax.experimental import pallas as pl
from jax.experimental.pallas import tpu as pltpu
from jax.experimental.pallas import tpu_sc as plsc

N = 10000
E = 160000
R = 8
D_EMB = 768
D_IN = 256
H = 16

NC, NS, LANES = 2, 16, 16
NW = NC * NS
CH = 128
EPW = 5120
NCHUNK = EPW // CH
EP = EPW * NW
TCHUNKS = EP // CH
NKEY = 81920
NPA = 10240

_AGG_TPW = NPA // NS
_KEY_TPW = NKEY // NS


def _deg_body(edges_hbm, zeros1_hbm, recip_out, edg_b, kidx_b, ones_b, rbuf,
              se0, se1, ss0, ss1, deg_sp):
    sid = lax.axis_index("s")
    ones_v = jnp.ones((LANES,), jnp.float32)
    nchunk = TCHUNKS // NS
    se = (se0, se1)
    ss = (ss0, ss1)

    dslot0 = sid * _KEY_TPW
    pltpu.sync_copy(zeros1_hbm, deg_sp.at[pl.ds(dslot0, _KEY_TPW)])
    for g in range(CH // LANES):
        ones_b[pl.ds(g * LANES, LANES)] = ones_v
    plsc.subcore_barrier()

    def start_edges(j, b):
        pltpu.async_copy(edges_hbm.at[sid * nchunk + j], edg_b.at[b], se[b])

    def wait_edges(b):
        pltpu.make_async_copy(edges_hbm.at[0], edg_b.at[b], se[b]).wait()

    def wait_scatter(b):
        pltpu.make_async_copy(ones_b, deg_sp.at[kidx_b.at[b]], ss[b]).wait()

    start_edges(0, 0)

    def _chunk(j, _):
        def it(b):
            ob = 1 - b
            wait_edges(b)
            @pl.when(j + 1 < nchunk)
            def _():
                start_edges(j + 1, ob)
            @pl.when(j >= 2)
            def _():
                wait_scatter(b)
            for g in range(CH // LANES):
                off = pl.multiple_of(g * LANES, LANES)
                d = edg_b[b, 1, pl.ds(off, LANES)]
                t = edg_b[b, 2, pl.ds(off, LANES)]
                kidx_b[b, pl.ds(off, LANES)] = d * R + t
            pltpu.async_copy(ones_b, deg_sp.at[kidx_b.at[b]], ss[b], add=True)
        @pl.when(lax.rem(j, 2) == 0)
        def _():
            it(0)
        @pl.when(lax.rem(j, 2) == 1)
        def _():
            it(1)
        return 0
    lax.fori_loop(0, nchunk, _chunk, 0)
    wait_scatter(0)
    wait_scatter(1)

    plsc.subcore_barrier()
    pltpu.sync_copy(deg_sp.at[pl.ds(dslot0, _KEY_TPW)], rbuf)

    def _recip(i, _):
        off = pl.multiple_of(i * LANES, LANES)
        rbuf[pl.ds(off, LANES)] = 1.0 / jnp.maximum(rbuf[pl.ds(off, LANES)],
                                                    1.0)
        return 0
    lax.fori_loop(0, _KEY_TPW // LANES, _recip, 0)
    pltpu.sync_copy(rbuf, recip_out.at[pl.ds(dslot0, _KEY_TPW)])


def _make_deg_pass():
    mesh = plsc.VectorSubcoreMesh(core_axis_name="c", subcore_axis_name="s",
                                  num_cores=1, num_subcores=NS)
    return pl.kernel(
        _deg_body,
        out_type=jax.ShapeDtypeStruct((NKEY,), jnp.float32),
        mesh=mesh,
        scratch_types=[
            pltpu.VMEM((2, 3, CH), jnp.int32),
            pltpu.VMEM((2, CH), jnp.int32),
            pltpu.VMEM((CH,), jnp.float32),
            pltpu.VMEM((_KEY_TPW,), jnp.float32),
            pltpu.SemaphoreType.DMA, pltpu.SemaphoreType.DMA,
            pltpu.SemaphoreType.DMA, pltpu.SemaphoreType.DMA,
            pltpu.VMEM_SHARED((NKEY,), jnp.float32),
        ],
        compiler_params=pltpu.CompilerParams(needs_layout_passes=False,
                                             use_tc_tiling_on_sc=False),
    )


def _edge_body(edges_hbm, z_hbm, recip_hbm, zeros_hbm, agg_out,
               edg_b, kidx_b, gkey_b, npad_b, recip_b, rows_b,
               se0, se1, sg0, sg1, ss0, ss1, agg_sp):
    cid = lax.axis_index("c")
    sid = lax.axis_index("s")
    wid = sid * NC + cid
    se, sg, ss = (se0, se1), (sg0, sg1), (ss0, ss1)

    pltpu.sync_copy(recip_hbm, recip_b)
    arow0 = sid * _AGG_TPW
    pltpu.sync_copy(zeros_hbm, agg_sp.at[pl.ds(arow0, _AGG_TPW)])
    plsc.subcore_barrier()

    def start_edges(j, b):
        pltpu.async_copy(edges_hbm.at[wid * NCHUNK + j], edg_b.at[b], se[b])

    def wait_edges(b):
        pltpu.make_async_copy(edges_hbm.at[0], edg_b.at[b], se[b]).wait()

    def compute_keys(b):
        for g in range(CH // LANES):
            off = pl.multiple_of(g * LANES, LANES)
            s = edg_b[b, 0, pl.ds(off, LANES)]
            d = edg_b[b, 1, pl.ds(off, LANES)]
            t = edg_b[b, 2, pl.ds(off, LANES)]
            kidx_b[b, pl.ds(off, LANES)] = d
            gkey_b[b, pl.ds(off, LANES)] = s * R + t
            nrm = plsc.load_gather(recip_b, [d * R + t])
            npad_b[b, pl.ds(LANES + off, LANES)] = nrm

    def start_gather(b):
        pltpu.async_copy(z_hbm.at[gkey_b.at[b]], rows_b.at[b], sg[b])

    def wait_gather(b):
        pltpu.make_async_copy(z_hbm.at[gkey_b.at[b]], rows_b.at[b],
                              sg[b]).wait()

    def compute_msgs(b):
        for l in range(CH):
            nspl = plsc.load_gather(
                npad_b.at[b], [jnp.full((LANES,), LANES + l, jnp.int32)])
            rows_b[b, l, :] = rows_b[b, l, :] * nspl

    def start_scatter(b):
        pltpu.async_copy(rows_b.at[b], agg_sp.at[kidx_b.at[b]], ss[b],
                         add=True)

    def wait_scatter(b):
        pltpu.make_async_copy(rows_b.at[b], agg_sp.at[kidx_b.at[b]],
                              ss[b]).wait()

    start_edges(jnp.int32(0), 0)
    wait_edges(0)
    compute_keys(0)
    start_gather(0)
    start_edges(jnp.int32(1), 1)

    def _chunk(j, _):
        def it(b):
            ob = 1 - b
            wait_gather(b)
            compute_msgs(b)
            start_scatter(b)
            @pl.when(j + 1 < NCHUNK)
            def _():
                wait_edges(ob)
                @pl.when(j >= 1)
                def _():
                    wait_scatter(ob)
                compute_keys(ob)
                start_gather(ob)
                @pl.when(j + 2 < NCHUNK)
                def _():
                    start_edges(j + 2, b)
        @pl.when(lax.rem(j, 2) == 0)
        def _():
            it(0)
        @pl.when(lax.rem(j, 2) == 1)
        def _():
            it(1)
        return 0
    lax.fori_loop(0, NCHUNK, _chunk, 0)
    wait_scatter(0)
    wait_scatter(1)

    plsc.subcore_barrier()
    pltpu.sync_copy(agg_sp.at[pl.ds(arow0, _AGG_TPW)],
                    agg_out.at[cid, pl.ds(arow0, _AGG_TPW)])


def _make_edge_pass():
    mesh = plsc.VectorSubcoreMesh(core_axis_name="c", subcore_axis_name="s",
                                  num_cores=NC, num_subcores=NS)
    return pl.kernel(
        _edge_body,
        out_type=jax.ShapeDtypeStruct((NC, NPA, LANES), jnp.float32),
        mesh=mesh,
        scratch_types=[
            pltpu.VMEM((2, 3, CH), jnp.int32),
            pltpu.VMEM((2, CH), jnp.int32),
            pltpu.VMEM((2, CH), jnp.int32),
            pltpu.VMEM((2, CH + LANES), jnp.float32),
            pltpu.VMEM((NKEY,), jnp.float32),
            pltpu.VMEM((2, CH, LANES), jnp.float32),
            pltpu.SemaphoreType.DMA, pltpu.SemaphoreType.DMA,
            pltpu.SemaphoreType.DMA, pltpu.SemaphoreType.DMA,
            pltpu.SemaphoreType.DMA, pltpu.SemaphoreType.DMA,
            pltpu.VMEM_SHARED((NPA, LANES), jnp.float32),
        ],
        compiler_params=pltpu.CompilerParams(needs_layout_passes=False,
                                             use_tc_tiling_on_sc=False),
    )


_RB = 1000
_DB = NKEY // 128 // 10


def _tc1_body(emb, wfc, bfc, wcat, root, bias, z_out, rp_out):
    h = jnp.dot(emb[...], wfc[...], preferred_element_type=jnp.float32) + bfc[...]
    z_out[...] = jnp.dot(h, wcat[...], preferred_element_type=jnp.float32)
    rp_out[...] = jnp.dot(h, root[...], preferred_element_type=jnp.float32) + bias[...]


def _tc1(emb, wfc, bfc, wcat, root, bias):
    f32 = jnp.float32
    full = lambda shape: pl.BlockSpec(shape, lambda i: (0, 0))
    return pl.pallas_call(
        _tc1_body,
        grid=(N // _RB,),
        in_specs=[
            pl.BlockSpec((_RB, D_EMB), lambda i: (i, 0)),
            full((D_EMB, D_IN)), full((1, D_IN)),
            full((D_IN, R * H)), full((D_IN, H)), full((1, H)),
        ],
        out_specs=[pl.BlockSpec((_RB, R * H), lambda i: (i, 0)),
                   pl.BlockSpec((_RB, H), lambda i: (i, 0))],
        out_shape=[jax.ShapeDtypeStruct((N, R * H), f32),
                   jax.ShapeDtypeStruct((N, H), f32)],
    )(emb, wfc, bfc, wcat, root, bias)


def _tc2_body(a0, a1, rp1, wcat2, root2, bias2, z2_out, rp2_out):
    x1 = jnp.maximum(a0[...] + a1[...] + rp1[...], 0.0)
    z2_out[...] = jnp.dot(x1, wcat2[...], preferred_element_type=jnp.float32)
    rp2_out[...] = jnp.dot(x1, root2[...], preferred_element_type=jnp.float32) + bias2[...]


def _tc2(a0, a1, rp1, wcat2, root2, bias2):
    f32 = jnp.float32
    full = lambda shape: pl.BlockSpec(shape, lambda i: (0, 0))
    blk = lambda w: pl.BlockSpec((_RB, w), lambda i: (i, 0))
    return pl.pallas_call(
        _tc2_body,
        grid=(N // _RB,),
        in_specs=[blk(H), blk(H), blk(H),
                  full((H, R * H)), full((H, H)), full((1, H))],
        out_specs=[blk(R * H), blk(H)],
        out_shape=[jax.ShapeDtypeStruct((N, R * H), f32),
                   jax.ShapeDtypeStruct((N, H), f32)],
    )(a0, a1, rp1, wcat2, root2, bias2)


def _tc3_body(a0, a1, rp2, out):
    out[...] = a0[...] + a1[...] + rp2[...]


def _tc3(a0, a1, rp2):
    blk = lambda w: pl.BlockSpec((_RB, w), lambda i: (i, 0))
    return pl.pallas_call(
        _tc3_body,
        grid=(N // _RB,),
        in_specs=[blk(H), blk(H), blk(H)],
        out_specs=blk(H),
        out_shape=jax.ShapeDtypeStruct((N, H), jnp.float32),
    )(a0, a1, rp2)


def kernel(embeddings, edge_index, edge_type, W_fc, b_fc,
           bases1, comp1, root1, bias1, bases2, comp2, root2, bias2):
    f32 = jnp.float32
    src = edge_index[0].astype(jnp.int32)
    dst = edge_index[1].astype(jnp.int32)
    et = edge_type.astype(jnp.int32)
    npad = EP - E
    src_p = jnp.concatenate([src, jnp.zeros((npad,), jnp.int32)])
    dst_p = jnp.concatenate([dst, jnp.full((npad,), N, jnp.int32)])
    et_p = jnp.concatenate([et, jnp.zeros((npad,), jnp.int32)])
    edges3 = jnp.stack([src_p.reshape(TCHUNKS, CH),
                        dst_p.reshape(TCHUNKS, CH),
                        et_p.reshape(TCHUNKS, CH)], axis=1)

    wcat1 = jnp.einsum('rb,bio->rio', comp1, bases1).transpose(1, 0, 2).reshape(D_IN, R * H)
    wcat2 = jnp.einsum('rb,bio->rio', comp2, bases2).transpose(1, 0, 2).reshape(H, R * H)
    zeros_hbm = jnp.zeros((_AGG_TPW, LANES), f32)
    zeros1_hbm = jnp.zeros((_KEY_TPW,), f32)

    recip_flat = _make_deg_pass()(edges3, zeros1_hbm)

    z1, rp1 = _tc1(embeddings, W_fc, b_fc.reshape(1, D_IN), wcat1,
                   root1, bias1.reshape(1, H))

    edge_pass = _make_edge_pass()
    agg1 = edge_pass(edges3, z1.reshape(N * R, LANES), recip_flat, zeros_hbm)
    a10 = agg1[0, :N]
    a11 = agg1[1, :N]

    z2, rp2 = _tc2(a10, a11, rp1, wcat2, root2, bias2.reshape(1, H))

    agg2 = edge_pass(edges3, z2.reshape(N * R, LANES), recip_flat, zeros_hbm)

    return _tc3(agg2[0, :N], agg2[1, :N], rp2)


# speedup vs baseline: 9.8898x; 1.0607x over previous
import functools
import numpy as np
import jax
import jax.numpy as jnp
from jax import lax
from jax.experimental import pallas as pl
from jax.experimental.pallas import tpu as pltpu
from jax.experimental.pallas import tpu_sc as plsc

N = 10000
E = 160000
R = 8
D_EMB = 768
D_IN = 256
H = 16

NC, NS, LANES = 2, 16, 16
NW = NC * NS
CH = 256
EPW = 5120
NCHUNK = EPW // CH
EP = EPW * NW
TCHUNKS = EP // CH
NKEY = 81920
NPA = 10240

_AGG_TPW = NPA // NS
_KEY_TPW = NKEY // NS


def _deg_body(edges_hbm, zeros1_hbm, recip_out, edg_b, kidx_b, ones_b, rbuf,
              se0, se1, ss0, ss1, deg_sp):
    sid = lax.axis_index("s")
    ones_v = jnp.ones((LANES,), jnp.float32)
    nchunk = TCHUNKS // NS
    se = (se0, se1)
    ss = (ss0, ss1)

    dslot0 = sid * _KEY_TPW
    pltpu.sync_copy(zeros1_hbm, deg_sp.at[pl.ds(dslot0, _KEY_TPW)])
    for g in range(CH // LANES):
        ones_b[pl.ds(g * LANES, LANES)] = ones_v
    plsc.subcore_barrier()

    def start_edges(j, b):
        pltpu.async_copy(edges_hbm.at[sid * nchunk + j], edg_b.at[b], se[b])

    def wait_edges(b):
        pltpu.make_async_copy(edges_hbm.at[0], edg_b.at[b], se[b]).wait()

    def wait_scatter(b):
        pltpu.make_async_copy(ones_b, deg_sp.at[kidx_b.at[b]], ss[b]).wait()

    start_edges(0, 0)

    def _chunk(j, _):
        def it(b):
            ob = 1 - b
            wait_edges(b)
            @pl.when(j + 1 < nchunk)
            def _():
                start_edges(j + 1, ob)
            @pl.when(j >= 2)
            def _():
                wait_scatter(b)
            for g in range(CH // LANES):
                off = pl.multiple_of(g * LANES, LANES)
                d = edg_b[b, 1, pl.ds(off, LANES)]
                t = edg_b[b, 2, pl.ds(off, LANES)]
                kidx_b[b, pl.ds(off, LANES)] = d * R + t
            pltpu.async_copy(ones_b, deg_sp.at[kidx_b.at[b]], ss[b], add=True)
        @pl.when(lax.rem(j, 2) == 0)
        def _():
            it(0)
        @pl.when(lax.rem(j, 2) == 1)
        def _():
            it(1)
        return 0
    lax.fori_loop(0, nchunk, _chunk, 0)
    wait_scatter(0)
    wait_scatter(1)

    plsc.subcore_barrier()
    pltpu.sync_copy(deg_sp.at[pl.ds(dslot0, _KEY_TPW)], rbuf)

    def _recip(i, _):
        off = pl.multiple_of(i * LANES, LANES)
        rbuf[pl.ds(off, LANES)] = 1.0 / jnp.maximum(rbuf[pl.ds(off, LANES)],
                                                    1.0)
        return 0
    lax.fori_loop(0, _KEY_TPW // LANES, _recip, 0)
    pltpu.sync_copy(rbuf, recip_out.at[pl.ds(dslot0, _KEY_TPW)])


def _make_deg_pass():
    mesh = plsc.VectorSubcoreMesh(core_axis_name="c", subcore_axis_name="s",
                                  num_cores=1, num_subcores=NS)
    return pl.kernel(
        _deg_body,
        out_type=jax.ShapeDtypeStruct((NKEY,), jnp.float32),
        mesh=mesh,
        scratch_types=[
            pltpu.VMEM((2, 3, CH), jnp.int32),
            pltpu.VMEM((2, CH), jnp.int32),
            pltpu.VMEM((CH,), jnp.float32),
            pltpu.VMEM((_KEY_TPW,), jnp.float32),
            pltpu.SemaphoreType.DMA, pltpu.SemaphoreType.DMA,
            pltpu.SemaphoreType.DMA, pltpu.SemaphoreType.DMA,
            pltpu.VMEM_SHARED((NKEY,), jnp.float32),
        ],
        compiler_params=pltpu.CompilerParams(needs_layout_passes=False,
                                             use_tc_tiling_on_sc=False),
    )


def _edge_body(edges_hbm, z_hbm, recip_hbm, zeros_hbm, agg_out,
               edg_b, kidx_b, gkey_b, npad_b, recip_b, rows_b,
               se0, se1, sg0, sg1, ss0, ss1, agg_sp):
    cid = lax.axis_index("c")
    sid = lax.axis_index("s")
    wid = sid * NC + cid
    se, sg, ss = (se0, se1), (sg0, sg1), (ss0, ss1)

    pltpu.sync_copy(recip_hbm, recip_b)
    arow0 = sid * _AGG_TPW
    pltpu.sync_copy(zeros_hbm, agg_sp.at[pl.ds(arow0, _AGG_TPW)])
    plsc.subcore_barrier()

    def start_edges(j, b):
        pltpu.async_copy(edges_hbm.at[wid * NCHUNK + j], edg_b.at[b], se[b])

    def wait_edges(b):
        pltpu.make_async_copy(edges_hbm.at[0], edg_b.at[b], se[b]).wait()

    def compute_keys(b):
        for g in range(CH // LANES):
            off = pl.multiple_of(g * LANES, LANES)
            s = edg_b[b, 0, pl.ds(off, LANES)]
            d = edg_b[b, 1, pl.ds(off, LANES)]
            t = edg_b[b, 2, pl.ds(off, LANES)]
            kidx_b[b, pl.ds(off, LANES)] = d
            gkey_b[b, pl.ds(off, LANES)] = s * R + t
            nrm = plsc.load_gather(recip_b, [d * R + t])
            npad_b[b, pl.ds(LANES + off, LANES)] = nrm

    def start_gather(b):
        pltpu.async_copy(z_hbm.at[gkey_b.at[b]], rows_b.at[b], sg[b])

    def wait_gather(b):
        pltpu.make_async_copy(z_hbm.at[gkey_b.at[b]], rows_b.at[b],
                              sg[b]).wait()

    def compute_msgs(b):
        for l in range(CH):
            nspl = plsc.load_gather(
                npad_b.at[b], [jnp.full((LANES,), LANES + l, jnp.int32)])
            rows_b[b, l, :] = rows_b[b, l, :] * nspl

    def start_scatter(b):
        pltpu.async_copy(rows_b.at[b], agg_sp.at[kidx_b.at[b]], ss[b],
                         add=True)

    def wait_scatter(b):
        pltpu.make_async_copy(rows_b.at[b], agg_sp.at[kidx_b.at[b]],
                              ss[b]).wait()

    start_edges(jnp.int32(0), 0)
    wait_edges(0)
    compute_keys(0)
    start_gather(0)
    start_edges(jnp.int32(1), 1)

    def _chunk(j, _):
        def it(b):
            ob = 1 - b
            wait_gather(b)
            compute_msgs(b)
            start_scatter(b)
            @pl.when(j + 1 < NCHUNK)
            def _():
                wait_edges(ob)
                @pl.when(j >= 1)
                def _():
                    wait_scatter(ob)
                compute_keys(ob)
                start_gather(ob)
                @pl.when(j + 2 < NCHUNK)
                def _():
                    start_edges(j + 2, b)
        @pl.when(lax.rem(j, 2) == 0)
        def _():
            it(0)
        @pl.when(lax.rem(j, 2) == 1)
        def _():
            it(1)
        return 0
    lax.fori_loop(0, NCHUNK, _chunk, 0)
    wait_scatter(0)
    wait_scatter(1)

    plsc.subcore_barrier()
    pltpu.sync_copy(agg_sp.at[pl.ds(arow0, _AGG_TPW)],
                    agg_out.at[cid, pl.ds(arow0, _AGG_TPW)])


def _make_edge_pass():
    mesh = plsc.VectorSubcoreMesh(core_axis_name="c", subcore_axis_name="s",
                                  num_cores=NC, num_subcores=NS)
    return pl.kernel(
        _edge_body,
        out_type=jax.ShapeDtypeStruct((NC, NPA, LANES), jnp.float32),
        mesh=mesh,
        scratch_types=[
            pltpu.VMEM((2, 3, CH), jnp.int32),
            pltpu.VMEM((2, CH), jnp.int32),
            pltpu.VMEM((2, CH), jnp.int32),
            pltpu.VMEM((2, CH + LANES), jnp.float32),
            pltpu.VMEM((NKEY,), jnp.float32),
            pltpu.VMEM((2, CH, LANES), jnp.float32),
            pltpu.SemaphoreType.DMA, pltpu.SemaphoreType.DMA,
            pltpu.SemaphoreType.DMA, pltpu.SemaphoreType.DMA,
            pltpu.SemaphoreType.DMA, pltpu.SemaphoreType.DMA,
            pltpu.VMEM_SHARED((NPA, LANES), jnp.float32),
        ],
        compiler_params=pltpu.CompilerParams(needs_layout_passes=False,
                                             use_tc_tiling_on_sc=False),
    )


_RB = 1000
_DB = NKEY // 128 // 10


def _tc1_body(emb, wfc, bfc, wcat, root, bias, z_out, rp_out):
    h = jnp.dot(emb[...], wfc[...], preferred_element_type=jnp.float32) + bfc[...]
    z_out[...] = jnp.dot(h, wcat[...], preferred_element_type=jnp.float32)
    rp_out[...] = jnp.dot(h, root[...], preferred_element_type=jnp.float32) + bias[...]


def _tc1(emb, wfc, bfc, wcat, root, bias):
    f32 = jnp.float32
    full = lambda shape: pl.BlockSpec(shape, lambda i: (0, 0))
    return pl.pallas_call(
        _tc1_body,
        grid=(N // _RB,),
        in_specs=[
            pl.BlockSpec((_RB, D_EMB), lambda i: (i, 0)),
            full((D_EMB, D_IN)), full((1, D_IN)),
            full((D_IN, R * H)), full((D_IN, H)), full((1, H)),
        ],
        out_specs=[pl.BlockSpec((_RB, R * H), lambda i: (i, 0)),
                   pl.BlockSpec((_RB, H), lambda i: (i, 0))],
        out_shape=[jax.ShapeDtypeStruct((N, R * H), f32),
                   jax.ShapeDtypeStruct((N, H), f32)],
    )(emb, wfc, bfc, wcat, root, bias)


def _tc2_body(a0, a1, rp1, wcat2, root2, bias2, z2_out, rp2_out):
    x1 = jnp.maximum(a0[...] + a1[...] + rp1[...], 0.0)
    z2_out[...] = jnp.dot(x1, wcat2[...], preferred_element_type=jnp.float32)
    rp2_out[...] = jnp.dot(x1, root2[...], preferred_element_type=jnp.float32) + bias2[...]


def _tc2(a0, a1, rp1, wcat2, root2, bias2):
    f32 = jnp.float32
    full = lambda shape: pl.BlockSpec(shape, lambda i: (0, 0))
    blk = lambda w: pl.BlockSpec((_RB, w), lambda i: (i, 0))
    return pl.pallas_call(
        _tc2_body,
        grid=(N // _RB,),
        in_specs=[blk(H), blk(H), blk(H),
                  full((H, R * H)), full((H, H)), full((1, H))],
        out_specs=[blk(R * H), blk(H)],
        out_shape=[jax.ShapeDtypeStruct((N, R * H), f32),
                   jax.ShapeDtypeStruct((N, H), f32)],
    )(a0, a1, rp1, wcat2, root2, bias2)


def _tc3_body(a0, a1, rp2, out):
    out[...] = a0[...] + a1[...] + rp2[...]


def _tc3(a0, a1, rp2):
    blk = lambda w: pl.BlockSpec((_RB, w), lambda i: (i, 0))
    return pl.pallas_call(
        _tc3_body,
        grid=(N // _RB,),
        in_specs=[blk(H), blk(H), blk(H)],
        out_specs=blk(H),
        out_shape=jax.ShapeDtypeStruct((N, H), jnp.float32),
    )(a0, a1, rp2)


def kernel(embeddings, edge_index, edge_type, W_fc, b_fc,
           bases1, comp1, root1, bias1, bases2, comp2, root2, bias2):
    f32 = jnp.float32
    src = edge_index[0].astype(jnp.int32)
    dst = edge_index[1].astype(jnp.int32)
    et = edge_type.astype(jnp.int32)
    npad = EP - E
    src_p = jnp.concatenate([src, jnp.zeros((npad,), jnp.int32)])
    dst_p = jnp.concatenate([dst, jnp.full((npad,), N, jnp.int32)])
    et_p = jnp.concatenate([et, jnp.zeros((npad,), jnp.int32)])
    edges3 = jnp.stack([src_p.reshape(TCHUNKS, CH),
                        dst_p.reshape(TCHUNKS, CH),
                        et_p.reshape(TCHUNKS, CH)], axis=1)

    wcat1 = jnp.einsum('rb,bio->rio', comp1, bases1).transpose(1, 0, 2).reshape(D_IN, R * H)
    wcat2 = jnp.einsum('rb,bio->rio', comp2, bases2).transpose(1, 0, 2).reshape(H, R * H)
    zeros_hbm = jnp.zeros((_AGG_TPW, LANES), f32)
    zeros1_hbm = jnp.zeros((_KEY_TPW,), f32)

    recip_flat = _make_deg_pass()(edges3, zeros1_hbm)

    z1, rp1 = _tc1(embeddings, W_fc, b_fc.reshape(1, D_IN), wcat1,
                   root1, bias1.reshape(1, H))

    edge_pass = _make_edge_pass()
    agg1 = edge_pass(edges3, z1.reshape(N * R, LANES), recip_flat, zeros_hbm)
    a10 = agg1[0, :N]
    a11 = agg1[1, :N]

    z2, rp2 = _tc2(a10, a11, rp1, wcat2, root2, bias2.reshape(1, H))

    agg2 = edge_pass(edges3, z2.reshape(N * R, LANES), recip_flat, zeros_hbm)

    return _tc3(agg2[0, :N], agg2[1, :N], rp2)


# speedup vs baseline: 10.1452x; 1.0258x over previous
import functools
import numpy as np
import jax
import jax.numpy as jnp
from jax import lax
from jax.experimental import pallas as pl
from jax.experimental.pallas import tpu as pltpu
from jax.experimental.pallas import tpu_sc as plsc

N = 10000
E = 160000
R = 8
D_EMB = 768
D_IN = 256
H = 16

NC, NS, LANES = 2, 16, 16
NW = NC * NS
CH = 512
EPW = 5120
NCHUNK = EPW // CH
EP = EPW * NW
TCHUNKS = EP // CH
NKEY = 81920
NPA = 10240

_AGG_TPW = NPA // NS
_KEY_TPW = NKEY // NS


def _deg_body(edges_hbm, zeros1_hbm, recip_out, edg_b, kidx_b, ones_b, rbuf,
              se0, se1, ss0, ss1, deg_sp):
    sid = lax.axis_index("s")
    ones_v = jnp.ones((LANES,), jnp.float32)
    nchunk = TCHUNKS // NS
    se = (se0, se1)
    ss = (ss0, ss1)

    dslot0 = sid * _KEY_TPW
    pltpu.sync_copy(zeros1_hbm, deg_sp.at[pl.ds(dslot0, _KEY_TPW)])
    for g in range(CH // LANES):
        ones_b[pl.ds(g * LANES, LANES)] = ones_v
    plsc.subcore_barrier()

    def start_edges(j, b):
        pltpu.async_copy(edges_hbm.at[sid * nchunk + j], edg_b.at[b], se[b])

    def wait_edges(b):
        pltpu.make_async_copy(edges_hbm.at[0], edg_b.at[b], se[b]).wait()

    def wait_scatter(b):
        pltpu.make_async_copy(ones_b, deg_sp.at[kidx_b.at[b]], ss[b]).wait()

    start_edges(0, 0)

    def _chunk(j, _):
        def it(b):
            ob = 1 - b
            wait_edges(b)
            @pl.when(j + 1 < nchunk)
            def _():
                start_edges(j + 1, ob)
            @pl.when(j >= 2)
            def _():
                wait_scatter(b)
            for g in range(CH // LANES):
                off = pl.multiple_of(g * LANES, LANES)
                d = edg_b[b, 1, pl.ds(off, LANES)]
                t = edg_b[b, 2, pl.ds(off, LANES)]
                kidx_b[b, pl.ds(off, LANES)] = d * R + t
            pltpu.async_copy(ones_b, deg_sp.at[kidx_b.at[b]], ss[b], add=True)
        @pl.when(lax.rem(j, 2) == 0)
        def _():
            it(0)
        @pl.when(lax.rem(j, 2) == 1)
        def _():
            it(1)
        return 0
    lax.fori_loop(0, nchunk, _chunk, 0)
    wait_scatter(0)
    wait_scatter(1)

    plsc.subcore_barrier()
    pltpu.sync_copy(deg_sp.at[pl.ds(dslot0, _KEY_TPW)], rbuf)

    def _recip(i, _):
        off = pl.multiple_of(i * LANES, LANES)
        rbuf[pl.ds(off, LANES)] = 1.0 / jnp.maximum(rbuf[pl.ds(off, LANES)],
                                                    1.0)
        return 0
    lax.fori_loop(0, _KEY_TPW // LANES, _recip, 0)
    pltpu.sync_copy(rbuf, recip_out.at[pl.ds(dslot0, _KEY_TPW)])


def _make_deg_pass():
    mesh = plsc.VectorSubcoreMesh(core_axis_name="c", subcore_axis_name="s",
                                  num_cores=1, num_subcores=NS)
    return pl.kernel(
        _deg_body,
        out_type=jax.ShapeDtypeStruct((NKEY,), jnp.float32),
        mesh=mesh,
        scratch_types=[
            pltpu.VMEM((2, 3, CH), jnp.int32),
            pltpu.VMEM((2, CH), jnp.int32),
            pltpu.VMEM((CH,), jnp.float32),
            pltpu.VMEM((_KEY_TPW,), jnp.float32),
            pltpu.SemaphoreType.DMA, pltpu.SemaphoreType.DMA,
            pltpu.SemaphoreType.DMA, pltpu.SemaphoreType.DMA,
            pltpu.VMEM_SHARED((NKEY,), jnp.float32),
        ],
        compiler_params=pltpu.CompilerParams(needs_layout_passes=False,
                                             use_tc_tiling_on_sc=False),
    )


def _edge_body(edges_hbm, z_hbm, recip_hbm, zeros_hbm, agg_out,
               edg_b, kidx_b, gkey_b, npad_b, recip_b, rows_b,
               se0, se1, sg0, sg1, ss0, ss1, agg_sp):
    cid = lax.axis_index("c")
    sid = lax.axis_index("s")
    wid = sid * NC + cid
    se, sg, ss = (se0, se1), (sg0, sg1), (ss0, ss1)

    pltpu.sync_copy(recip_hbm, recip_b)
    arow0 = sid * _AGG_TPW
    pltpu.sync_copy(zeros_hbm, agg_sp.at[pl.ds(arow0, _AGG_TPW)])
    plsc.subcore_barrier()

    def start_edges(j, b):
        pltpu.async_copy(edges_hbm.at[wid * NCHUNK + j], edg_b.at[b], se[b])

    def wait_edges(b):
        pltpu.make_async_copy(edges_hbm.at[0], edg_b.at[b], se[b]).wait()

    def compute_keys(b):
        for g in range(CH // LANES):
            off = pl.multiple_of(g * LANES, LANES)
            s = edg_b[b, 0, pl.ds(off, LANES)]
            d = edg_b[b, 1, pl.ds(off, LANES)]
            t = edg_b[b, 2, pl.ds(off, LANES)]
            kidx_b[b, pl.ds(off, LANES)] = d
            gkey_b[b, pl.ds(off, LANES)] = s * R + t
            nrm = plsc.load_gather(recip_b, [d * R + t])
            npad_b[b, pl.ds(LANES + off, LANES)] = nrm

    def start_gather(b):
        pltpu.async_copy(z_hbm.at[gkey_b.at[b]], rows_b.at[b], sg[b])

    def wait_gather(b):
        pltpu.make_async_copy(z_hbm.at[gkey_b.at[b]], rows_b.at[b],
                              sg[b]).wait()

    def compute_msgs(b):
        for l in range(CH):
            nspl = plsc.load_gather(
                npad_b.at[b], [jnp.full((LANES,), LANES + l, jnp.int32)])
            rows_b[b, l, :] = rows_b[b, l, :] * nspl

    def start_scatter(b):
        pltpu.async_copy(rows_b.at[b], agg_sp.at[kidx_b.at[b]], ss[b],
                         add=True)

    def wait_scatter(b):
        pltpu.make_async_copy(rows_b.at[b], agg_sp.at[kidx_b.at[b]],
                              ss[b]).wait()

    start_edges(jnp.int32(0), 0)
    wait_edges(0)
    compute_keys(0)
    start_gather(0)
    start_edges(jnp.int32(1), 1)

    def _chunk(j, _):
        def it(b):
            ob = 1 - b
            wait_gather(b)
            compute_msgs(b)
            start_scatter(b)
            @pl.when(j + 1 < NCHUNK)
            def _():
                wait_edges(ob)
                @pl.when(j >= 1)
                def _():
                    wait_scatter(ob)
                compute_keys(ob)
                start_gather(ob)
                @pl.when(j + 2 < NCHUNK)
                def _():
                    start_edges(j + 2, b)
        @pl.when(lax.rem(j, 2) == 0)
        def _():
            it(0)
        @pl.when(lax.rem(j, 2) == 1)
        def _():
            it(1)
        return 0
    lax.fori_loop(0, NCHUNK, _chunk, 0)
    wait_scatter(0)
    wait_scatter(1)

    plsc.subcore_barrier()
    pltpu.sync_copy(agg_sp.at[pl.ds(arow0, _AGG_TPW)],
                    agg_out.at[cid, pl.ds(arow0, _AGG_TPW)])


def _make_edge_pass():
    mesh = plsc.VectorSubcoreMesh(core_axis_name="c", subcore_axis_name="s",
                                  num_cores=NC, num_subcores=NS)
    return pl.kernel(
        _edge_body,
        out_type=jax.ShapeDtypeStruct((NC, NPA, LANES), jnp.float32),
        mesh=mesh,
        scratch_types=[
            pltpu.VMEM((2, 3, CH), jnp.int32),
            pltpu.VMEM((2, CH), jnp.int32),
            pltpu.VMEM((2, CH), jnp.int32),
            pltpu.VMEM((2, CH + LANES), jnp.float32),
            pltpu.VMEM((NKEY,), jnp.float32),
            pltpu.VMEM((2, CH, LANES), jnp.float32),
            pltpu.SemaphoreType.DMA, pltpu.SemaphoreType.DMA,
            pltpu.SemaphoreType.DMA, pltpu.SemaphoreType.DMA,
            pltpu.SemaphoreType.DMA, pltpu.SemaphoreType.DMA,
            pltpu.VMEM_SHARED((NPA, LANES), jnp.float32),
        ],
        compiler_params=pltpu.CompilerParams(needs_layout_passes=False,
                                             use_tc_tiling_on_sc=False),
    )


_RB = 1000
_DB = NKEY // 128 // 10


def _tc1_body(emb, wfc, bfc, wcat, root, bias, z_out, rp_out):
    h = jnp.dot(emb[...], wfc[...], preferred_element_type=jnp.float32) + bfc[...]
    z_out[...] = jnp.dot(h, wcat[...], preferred_element_type=jnp.float32)
    rp_out[...] = jnp.dot(h, root[...], preferred_element_type=jnp.float32) + bias[...]


def _tc1(emb, wfc, bfc, wcat, root, bias):
    f32 = jnp.float32
    full = lambda shape: pl.BlockSpec(shape, lambda i: (0, 0))
    return pl.pallas_call(
        _tc1_body,
        grid=(N // _RB,),
        in_specs=[
            pl.BlockSpec((_RB, D_EMB), lambda i: (i, 0)),
            full((D_EMB, D_IN)), full((1, D_IN)),
            full((D_IN, R * H)), full((D_IN, H)), full((1, H)),
        ],
        out_specs=[pl.BlockSpec((_RB, R * H), lambda i: (i, 0)),
                   pl.BlockSpec((_RB, H), lambda i: (i, 0))],
        out_shape=[jax.ShapeDtypeStruct((N, R * H), f32),
                   jax.ShapeDtypeStruct((N, H), f32)],
    )(emb, wfc, bfc, wcat, root, bias)


def _tc2_body(a0, a1, rp1, wcat2, root2, bias2, z2_out, rp2_out):
    x1 = jnp.maximum(a0[...] + a1[...] + rp1[...], 0.0)
    z2_out[...] = jnp.dot(x1, wcat2[...], preferred_element_type=jnp.float32)
    rp2_out[...] = jnp.dot(x1, root2[...], preferred_element_type=jnp.float32) + bias2[...]


def _tc2(a0, a1, rp1, wcat2, root2, bias2):
    f32 = jnp.float32
    full = lambda shape: pl.BlockSpec(shape, lambda i: (0, 0))
    blk = lambda w: pl.BlockSpec((_RB, w), lambda i: (i, 0))
    return pl.pallas_call(
        _tc2_body,
        grid=(N // _RB,),
        in_specs=[blk(H), blk(H), blk(H),
                  full((H, R * H)), full((H, H)), full((1, H))],
        out_specs=[blk(R * H), blk(H)],
        out_shape=[jax.ShapeDtypeStruct((N, R * H), f32),
                   jax.ShapeDtypeStruct((N, H), f32)],
    )(a0, a1, rp1, wcat2, root2, bias2)


def _tc3_body(a0, a1, rp2, out):
    out[...] = a0[...] + a1[...] + rp2[...]


def _tc3(a0, a1, rp2):
    blk = lambda w: pl.BlockSpec((_RB, w), lambda i: (i, 0))
    return pl.pallas_call(
        _tc3_body,
        grid=(N // _RB,),
        in_specs=[blk(H), blk(H), blk(H)],
        out_specs=blk(H),
        out_shape=jax.ShapeDtypeStruct((N, H), jnp.float32),
    )(a0, a1, rp2)


def kernel(embeddings, edge_index, edge_type, W_fc, b_fc,
           bases1, comp1, root1, bias1, bases2, comp2, root2, bias2):
    f32 = jnp.float32
    src = edge_index[0].astype(jnp.int32)
    dst = edge_index[1].astype(jnp.int32)
    et = edge_type.astype(jnp.int32)
    npad = EP - E
    src_p = jnp.concatenate([src, jnp.zeros((npad,), jnp.int32)])
    dst_p = jnp.concatenate([dst, jnp.full((npad,), N, jnp.int32)])
    et_p = jnp.concatenate([et, jnp.zeros((npad,), jnp.int32)])
    edges3 = jnp.stack([src_p.reshape(TCHUNKS, CH),
                        dst_p.reshape(TCHUNKS, CH),
                        et_p.reshape(TCHUNKS, CH)], axis=1)

    wcat1 = jnp.einsum('rb,bio->rio', comp1, bases1).transpose(1, 0, 2).reshape(D_IN, R * H)
    wcat2 = jnp.einsum('rb,bio->rio', comp2, bases2).transpose(1, 0, 2).reshape(H, R * H)
    zeros_hbm = jnp.zeros((_AGG_TPW, LANES), f32)
    zeros1_hbm = jnp.zeros((_KEY_TPW,), f32)

    recip_flat = _make_deg_pass()(edges3, zeros1_hbm)

    z1, rp1 = _tc1(embeddings, W_fc, b_fc.reshape(1, D_IN), wcat1,
                   root1, bias1.reshape(1, H))

    edge_pass = _make_edge_pass()
    agg1 = edge_pass(edges3, z1.reshape(N * R, LANES), recip_flat, zeros_hbm)
    a10 = agg1[0, :N]
    a11 = agg1[1, :N]

    z2, rp2 = _tc2(a10, a11, rp1, wcat2, root2, bias2.reshape(1, H))

    agg2 = edge_pass(edges3, z2.reshape(N * R, LANES), recip_flat, zeros_hbm)

    return _tc3(agg2[0, :N], agg2[1, :N], rp2)


# speedup vs baseline: 11.8400x; 1.1670x over previous
import functools
import numpy as np
import jax
import jax.numpy as jnp
from jax import lax
from jax.experimental import pallas as pl
from jax.experimental.pallas import tpu as pltpu
from jax.experimental.pallas import tpu_sc as plsc

N = 10000
E = 160000
R = 8
D_EMB = 768
D_IN = 256
H = 16

NC, NS, LANES = 2, 16, 16
NW = NC * NS
CH = 512
EPW = 5120
NCHUNK = EPW // CH
EP = EPW * NW
TCHUNKS = EP // CH
NKEY = 81920
NPA = 10112

_AGG_TPW = NPA // NS
_KEY_TPW = NKEY // NS


def _deg_body(edges_hbm, zeros1_hbm, recip_out, edg_b, kidx_b, ones_b, rbuf,
              se0, se1, ss0, ss1, deg_sp):
    sid = lax.axis_index("s")
    ones_v = jnp.ones((LANES,), jnp.float32)
    nchunk = TCHUNKS // NS
    se = (se0, se1)
    ss = (ss0, ss1)

    dslot0 = sid * _KEY_TPW
    pltpu.sync_copy(zeros1_hbm, deg_sp.at[pl.ds(dslot0, _KEY_TPW)])
    for g in range(CH // LANES):
        ones_b[pl.ds(g * LANES, LANES)] = ones_v
    plsc.subcore_barrier()

    def start_edges(j, b):
        pltpu.async_copy(edges_hbm.at[sid * nchunk + j], edg_b.at[b], se[b])

    def wait_edges(b):
        pltpu.make_async_copy(edges_hbm.at[0], edg_b.at[b], se[b]).wait()

    def wait_scatter(b):
        pltpu.make_async_copy(ones_b, deg_sp.at[kidx_b.at[b]], ss[b]).wait()

    start_edges(0, 0)

    def _chunk(j, _):
        def it(b):
            ob = 1 - b
            wait_edges(b)
            @pl.when(j + 1 < nchunk)
            def _():
                start_edges(j + 1, ob)
            @pl.when(j >= 2)
            def _():
                wait_scatter(b)
            for g in range(CH // LANES):
                off = pl.multiple_of(g * LANES, LANES)
                d = edg_b[b, 1, pl.ds(off, LANES)]
                t = edg_b[b, 2, pl.ds(off, LANES)]
                kidx_b[b, pl.ds(off, LANES)] = d * R + t
            pltpu.async_copy(ones_b, deg_sp.at[kidx_b.at[b]], ss[b], add=True)
        @pl.when(lax.rem(j, 2) == 0)
        def _():
            it(0)
        @pl.when(lax.rem(j, 2) == 1)
        def _():
            it(1)
        return 0
    lax.fori_loop(0, nchunk, _chunk, 0)
    wait_scatter(0)
    wait_scatter(1)

    plsc.subcore_barrier()
    pltpu.sync_copy(deg_sp.at[pl.ds(dslot0, _KEY_TPW)], rbuf)

    def _recip(i, _):
        off = pl.multiple_of(i * LANES, LANES)
        rbuf[pl.ds(off, LANES)] = 1.0 / jnp.maximum(rbuf[pl.ds(off, LANES)],
                                                    1.0)
        return 0
    lax.fori_loop(0, _KEY_TPW // LANES, _recip, 0)
    pltpu.sync_copy(rbuf, recip_out.at[pl.ds(dslot0, _KEY_TPW)])


def _make_deg_pass():
    mesh = plsc.VectorSubcoreMesh(core_axis_name="c", subcore_axis_name="s",
                                  num_cores=1, num_subcores=NS)
    return pl.kernel(
        _deg_body,
        out_type=jax.ShapeDtypeStruct((NKEY,), jnp.float32),
        mesh=mesh,
        scratch_types=[
            pltpu.VMEM((2, 3, CH), jnp.int32),
            pltpu.VMEM((2, CH), jnp.int32),
            pltpu.VMEM((CH,), jnp.float32),
            pltpu.VMEM((_KEY_TPW,), jnp.float32),
            pltpu.SemaphoreType.DMA, pltpu.SemaphoreType.DMA,
            pltpu.SemaphoreType.DMA, pltpu.SemaphoreType.DMA,
            pltpu.VMEM_SHARED((NKEY,), jnp.float32),
        ],
        compiler_params=pltpu.CompilerParams(needs_layout_passes=False,
                                             use_tc_tiling_on_sc=False),
    )


def _edge_body(edges_hbm, z_hbm, recip_hbm, zeros_hbm, agg_out,
               edg_b, kidx_b, gkey_b, npad_b, recip_b, rows_b, msg_b,
               se0, se1, sg0, sg1, ss0, ss1, agg_sp):
    cid = lax.axis_index("c")
    sid = lax.axis_index("s")
    wid = sid * NC + cid
    se, sg, ss = (se0, se1), (sg0, sg1), (ss0, ss1)

    pltpu.sync_copy(recip_hbm, recip_b)
    arow0 = sid * _AGG_TPW
    pltpu.sync_copy(zeros_hbm, agg_sp.at[pl.ds(arow0, _AGG_TPW)])
    plsc.subcore_barrier()

    def start_edges(j, b):
        pltpu.async_copy(edges_hbm.at[wid * NCHUNK + j], edg_b.at[b], se[b])

    def wait_edges(b):
        pltpu.make_async_copy(edges_hbm.at[0], edg_b.at[b], se[b]).wait()

    def compute_gkeys(b):
        for g in range(CH // LANES):
            off = pl.multiple_of(g * LANES, LANES)
            s = edg_b[b, 0, pl.ds(off, LANES)]
            t = edg_b[b, 2, pl.ds(off, LANES)]
            gkey_b[b, pl.ds(off, LANES)] = s * R + t
            d = edg_b[b, 1, pl.ds(off, LANES)]
            nrm = plsc.load_gather(recip_b, [d * R + t])
            npad_b[b, pl.ds(LANES + off, LANES)] = nrm

    def compute_kidx(b):
        for g in range(CH // LANES):
            off = pl.multiple_of(g * LANES, LANES)
            kidx_b[b, pl.ds(off, LANES)] = edg_b[b, 1, pl.ds(off, LANES)]

    def start_gather(b):
        pltpu.async_copy(z_hbm.at[gkey_b.at[b]], rows_b.at[b], sg[b])

    def wait_gather(b):
        pltpu.make_async_copy(z_hbm.at[gkey_b.at[b]], rows_b.at[b],
                              sg[b]).wait()

    def compute_msgs(b):
        for l in range(CH):
            nspl = plsc.load_gather(
                npad_b.at[b], [jnp.full((LANES,), LANES + l, jnp.int32)])
            msg_b[b, l, :] = rows_b[b, l, :] * nspl

    def start_scatter(b):
        pltpu.async_copy(msg_b.at[b], agg_sp.at[kidx_b.at[b]], ss[b],
                         add=True)

    def wait_scatter(b):
        pltpu.make_async_copy(msg_b.at[b], agg_sp.at[kidx_b.at[b]],
                              ss[b]).wait()

    start_edges(jnp.int32(0), 0)
    wait_edges(0)
    compute_gkeys(0)
    compute_kidx(0)
    start_gather(0)
    start_edges(jnp.int32(1), 1)

    def _chunk(j, _):
        def it(b):
            ob = 1 - b
            @pl.when(j + 1 < NCHUNK)
            def _():
                wait_edges(ob)
                compute_gkeys(ob)
                start_gather(ob)
                @pl.when(j + 2 < NCHUNK)
                def _():
                    start_edges(j + 2, b)
            wait_gather(b)
            compute_msgs(b)
            @pl.when(j + 1 < NCHUNK)
            def _():
                @pl.when(j >= 1)
                def _():
                    wait_scatter(ob)
                compute_kidx(ob)
            start_scatter(b)
        @pl.when(lax.rem(j, 2) == 0)
        def _():
            it(0)
        @pl.when(lax.rem(j, 2) == 1)
        def _():
            it(1)
        return 0
    lax.fori_loop(0, NCHUNK, _chunk, 0)
    wait_scatter(0)
    wait_scatter(1)

    plsc.subcore_barrier()
    pltpu.sync_copy(agg_sp.at[pl.ds(arow0, _AGG_TPW)],
                    agg_out.at[cid, pl.ds(arow0, _AGG_TPW)])


def _make_edge_pass():
    mesh = plsc.VectorSubcoreMesh(core_axis_name="c", subcore_axis_name="s",
                                  num_cores=NC, num_subcores=NS)
    return pl.kernel(
        _edge_body,
        out_type=jax.ShapeDtypeStruct((NC, NPA, LANES), jnp.float32),
        mesh=mesh,
        scratch_types=[
            pltpu.VMEM((2, 3, CH), jnp.int32),
            pltpu.VMEM((2, CH), jnp.int32),
            pltpu.VMEM((2, CH), jnp.int32),
            pltpu.VMEM((2, CH + LANES), jnp.float32),
            pltpu.VMEM((NKEY,), jnp.float32),
            pltpu.VMEM((2, CH, LANES), jnp.float32),
            pltpu.VMEM((2, CH, LANES), jnp.float32),
            pltpu.SemaphoreType.DMA, pltpu.SemaphoreType.DMA,
            pltpu.SemaphoreType.DMA, pltpu.SemaphoreType.DMA,
            pltpu.SemaphoreType.DMA, pltpu.SemaphoreType.DMA,
            pltpu.VMEM_SHARED((NPA, LANES), jnp.float32),
        ],
        compiler_params=pltpu.CompilerParams(needs_layout_passes=False,
                                             use_tc_tiling_on_sc=False),
    )


_RB = 1000
_DB = NKEY // 128 // 10


def _tc1_body(emb, wfc, bfc, wcat, root, bias, z_out, rp_out):
    h = jnp.dot(emb[...], wfc[...], preferred_element_type=jnp.float32) + bfc[...]
    z_out[...] = jnp.dot(h, wcat[...], preferred_element_type=jnp.float32)
    rp_out[...] = jnp.dot(h, root[...], preferred_element_type=jnp.float32) + bias[...]


def _tc1(emb, wfc, bfc, wcat, root, bias):
    f32 = jnp.float32
    full = lambda shape: pl.BlockSpec(shape, lambda i: (0, 0))
    return pl.pallas_call(
        _tc1_body,
        grid=(N // _RB,),
        in_specs=[
            pl.BlockSpec((_RB, D_EMB), lambda i: (i, 0)),
            full((D_EMB, D_IN)), full((1, D_IN)),
            full((D_IN, R * H)), full((D_IN, H)), full((1, H)),
        ],
        out_specs=[pl.BlockSpec((_RB, R * H), lambda i: (i, 0)),
                   pl.BlockSpec((_RB, H), lambda i: (i, 0))],
        out_shape=[jax.ShapeDtypeStruct((N, R * H), f32),
                   jax.ShapeDtypeStruct((N, H), f32)],
    )(emb, wfc, bfc, wcat, root, bias)


def _tc2_body(a0, a1, rp1, wcat2, root2, bias2, z2_out, rp2_out):
    x1 = jnp.maximum(a0[...] + a1[...] + rp1[...], 0.0)
    z2_out[...] = jnp.dot(x1, wcat2[...], preferred_element_type=jnp.float32)
    rp2_out[...] = jnp.dot(x1, root2[...], preferred_element_type=jnp.float32) + bias2[...]


def _tc2(a0, a1, rp1, wcat2, root2, bias2):
    f32 = jnp.float32
    full = lambda shape: pl.BlockSpec(shape, lambda i: (0, 0))
    blk = lambda w: pl.BlockSpec((_RB, w), lambda i: (i, 0))
    return pl.pallas_call(
        _tc2_body,
        grid=(N // _RB,),
        in_specs=[blk(H), blk(H), blk(H),
                  full((H, R * H)), full((H, H)), full((1, H))],
        out_specs=[blk(R * H), blk(H)],
        out_shape=[jax.ShapeDtypeStruct((N, R * H), f32),
                   jax.ShapeDtypeStruct((N, H), f32)],
    )(a0, a1, rp1, wcat2, root2, bias2)


def _tc3_body(a0, a1, rp2, out):
    out[...] = a0[...] + a1[...] + rp2[...]


def _tc3(a0, a1, rp2):
    blk = lambda w: pl.BlockSpec((_RB, w), lambda i: (i, 0))
    return pl.pallas_call(
        _tc3_body,
        grid=(N // _RB,),
        in_specs=[blk(H), blk(H), blk(H)],
        out_specs=blk(H),
        out_shape=jax.ShapeDtypeStruct((N, H), jnp.float32),
    )(a0, a1, rp2)


def kernel(embeddings, edge_index, edge_type, W_fc, b_fc,
           bases1, comp1, root1, bias1, bases2, comp2, root2, bias2):
    f32 = jnp.float32
    src = edge_index[0].astype(jnp.int32)
    dst = edge_index[1].astype(jnp.int32)
    et = edge_type.astype(jnp.int32)
    npad = EP - E
    src_p = jnp.concatenate([src, jnp.zeros((npad,), jnp.int32)])
    dst_p = jnp.concatenate([dst, jnp.full((npad,), N, jnp.int32)])
    et_p = jnp.concatenate([et, jnp.zeros((npad,), jnp.int32)])
    edges3 = jnp.stack([src_p.reshape(TCHUNKS, CH),
                        dst_p.reshape(TCHUNKS, CH),
                        et_p.reshape(TCHUNKS, CH)], axis=1)

    wcat1 = jnp.einsum('rb,bio->rio', comp1, bases1).transpose(1, 0, 2).reshape(D_IN, R * H)
    wcat2 = jnp.einsum('rb,bio->rio', comp2, bases2).transpose(1, 0, 2).reshape(H, R * H)
    zeros_hbm = jnp.zeros((_AGG_TPW, LANES), f32)
    zeros1_hbm = jnp.zeros((_KEY_TPW,), f32)

    recip_flat = _make_deg_pass()(edges3, zeros1_hbm)

    z1, rp1 = _tc1(embeddings, W_fc, b_fc.reshape(1, D_IN), wcat1,
                   root1, bias1.reshape(1, H))

    edge_pass = _make_edge_pass()
    agg1 = edge_pass(edges3, z1.reshape(N * R, LANES), recip_flat, zeros_hbm)
    a10 = agg1[0, :N]
    a11 = agg1[1, :N]

    z2, rp2 = _tc2(a10, a11, rp1, wcat2, root2, bias2.reshape(1, H))

    agg2 = edge_pass(edges3, z2.reshape(N * R, LANES), recip_flat, zeros_hbm)

    return _tc3(agg2[0, :N], agg2[1, :N], rp2)


# speedup vs baseline: 11.8837x; 1.0037x over previous
import functools
import numpy as np
import jax
import jax.numpy as jnp
from jax import lax
from jax.experimental import pallas as pl
from jax.experimental.pallas import tpu as pltpu
from jax.experimental.pallas import tpu_sc as plsc

N = 10000
E = 160000
R = 8
D_EMB = 768
D_IN = 256
H = 16

NC, NS, LANES = 2, 16, 16
NW = NC * NS
CH = 512
EPW = 5120
NCHUNK = EPW // CH
EP = EPW * NW
TCHUNKS = EP // CH
NKEY = 81920
NPA = 10112

_AGG_TPW = NPA // NS
_KEY_TPW = NKEY // NS


def _deg_body(edges_hbm, zeros1_hbm, recip_out, edg_b, kidx_b, ones_b, rbuf,
              se0, se1, ss0, ss1, deg_sp):
    sid = lax.axis_index("s")
    ones_v = jnp.ones((LANES,), jnp.float32)
    nchunk = TCHUNKS // NS
    se = (se0, se1)
    ss = (ss0, ss1)

    dslot0 = sid * _KEY_TPW
    pltpu.sync_copy(zeros1_hbm, deg_sp.at[pl.ds(dslot0, _KEY_TPW)])
    for g in range(CH // LANES):
        ones_b[pl.ds(g * LANES, LANES)] = ones_v
    plsc.subcore_barrier()

    def start_edges(j, b):
        pltpu.async_copy(edges_hbm.at[sid * nchunk + j], edg_b.at[b], se[b])

    def wait_edges(b):
        pltpu.make_async_copy(edges_hbm.at[0], edg_b.at[b], se[b]).wait()

    def wait_scatter(b):
        pltpu.make_async_copy(ones_b, deg_sp.at[kidx_b.at[b]], ss[b]).wait()

    start_edges(0, 0)

    def _chunk(j, _):
        def it(b):
            ob = 1 - b
            wait_edges(b)
            @pl.when(j + 1 < nchunk)
            def _():
                start_edges(j + 1, ob)
            @pl.when(j >= 2)
            def _():
                wait_scatter(b)
            for g in range(CH // LANES):
                off = pl.multiple_of(g * LANES, LANES)
                d = edg_b[b, 1, pl.ds(off, LANES)]
                t = edg_b[b, 2, pl.ds(off, LANES)]
                kidx_b[b, pl.ds(off, LANES)] = d * R + t
            pltpu.async_copy(ones_b, deg_sp.at[kidx_b.at[b]], ss[b], add=True)
        @pl.when(lax.rem(j, 2) == 0)
        def _():
            it(0)
        @pl.when(lax.rem(j, 2) == 1)
        def _():
            it(1)
        return 0
    lax.fori_loop(0, nchunk, _chunk, 0)
    wait_scatter(0)
    wait_scatter(1)

    plsc.subcore_barrier()
    pltpu.sync_copy(deg_sp.at[pl.ds(dslot0, _KEY_TPW)], rbuf)

    def _recip(i, _):
        off = pl.multiple_of(i * LANES, LANES)
        rbuf[pl.ds(off, LANES)] = 1.0 / jnp.maximum(rbuf[pl.ds(off, LANES)],
                                                    1.0)
        return 0
    lax.fori_loop(0, _KEY_TPW // LANES, _recip, 0)
    pltpu.sync_copy(rbuf, recip_out.at[pl.ds(dslot0, _KEY_TPW)])


def _make_deg_pass():
    mesh = plsc.VectorSubcoreMesh(core_axis_name="c", subcore_axis_name="s",
                                  num_cores=1, num_subcores=NS)
    return pl.kernel(
        _deg_body,
        out_type=jax.ShapeDtypeStruct((NKEY,), jnp.float32),
        mesh=mesh,
        scratch_types=[
            pltpu.VMEM((2, 3, CH), jnp.int32),
            pltpu.VMEM((2, CH), jnp.int32),
            pltpu.VMEM((CH,), jnp.float32),
            pltpu.VMEM((_KEY_TPW,), jnp.float32),
            pltpu.SemaphoreType.DMA, pltpu.SemaphoreType.DMA,
            pltpu.SemaphoreType.DMA, pltpu.SemaphoreType.DMA,
            pltpu.VMEM_SHARED((NKEY,), jnp.float32),
        ],
        compiler_params=pltpu.CompilerParams(needs_layout_passes=False,
                                             use_tc_tiling_on_sc=False),
    )


def _edge_body(edges_hbm, z_hbm, recip_hbm, zeros_hbm, agg_out,
               edg_b, kidx_b, gkey_b, npad_b, recip_b, rows_b, msg_b,
               se0, se1, sg0, sg1, ss0, ss1, agg_sp):
    cid = lax.axis_index("c")
    sid = lax.axis_index("s")
    wid = sid * NC + cid
    se, sg, ss = (se0, se1), (sg0, sg1), (ss0, ss1)

    pltpu.sync_copy(recip_hbm, recip_b)
    arow0 = sid * _AGG_TPW
    pltpu.sync_copy(zeros_hbm, agg_sp.at[pl.ds(arow0, _AGG_TPW)])
    plsc.subcore_barrier()

    def start_edges(j, b):
        pltpu.async_copy(edges_hbm.at[wid * NCHUNK + j], edg_b.at[b], se[b])

    def wait_edges(b):
        pltpu.make_async_copy(edges_hbm.at[0], edg_b.at[b], se[b]).wait()

    def compute_gkeys(b):
        for g in range(CH // LANES):
            off = pl.multiple_of(g * LANES, LANES)
            s = edg_b[b, 0, pl.ds(off, LANES)]
            t = edg_b[b, 2, pl.ds(off, LANES)]
            gkey_b[b, pl.ds(off, LANES)] = s * R + t
            d = edg_b[b, 1, pl.ds(off, LANES)]
            nrm = plsc.load_gather(recip_b, [d * R + t])
            npad_b[b, pl.ds(LANES + off, LANES)] = nrm

    def compute_kidx(b):
        for g in range(CH // LANES):
            off = pl.multiple_of(g * LANES, LANES)
            kidx_b[b, pl.ds(off, LANES)] = edg_b[b, 1, pl.ds(off, LANES)]

    def start_gather(b):
        pltpu.async_copy(z_hbm.at[gkey_b.at[b]], rows_b.at[b], sg[b])

    def wait_gather(b):
        pltpu.make_async_copy(z_hbm.at[gkey_b.at[b]], rows_b.at[b],
                              sg[b]).wait()

    def compute_msgs(b):
        for l in range(CH):
            nspl = plsc.load_gather(
                npad_b.at[b], [jnp.full((LANES,), LANES + l, jnp.int32)])
            msg_b[b, l, :] = rows_b[b, l, :] * nspl

    def start_scatter(b):
        pltpu.async_copy(msg_b.at[b], agg_sp.at[kidx_b.at[b]], ss[b],
                         add=True)

    def wait_scatter(b):
        pltpu.make_async_copy(msg_b.at[b], agg_sp.at[kidx_b.at[b]],
                              ss[b]).wait()

    start_edges(jnp.int32(0), 0)
    wait_edges(0)
    compute_gkeys(0)
    compute_kidx(0)
    start_gather(0)
    start_edges(jnp.int32(1), 1)

    def _chunk(j, _):
        def it(b):
            ob = 1 - b
            @pl.when(j + 1 < NCHUNK)
            def _():
                wait_edges(ob)
                compute_gkeys(ob)
                start_gather(ob)
                @pl.when(j + 2 < NCHUNK)
                def _():
                    start_edges(j + 2, b)
            wait_gather(b)
            compute_msgs(b)
            @pl.when(j + 1 < NCHUNK)
            def _():
                @pl.when(j >= 1)
                def _():
                    wait_scatter(ob)
                compute_kidx(ob)
            start_scatter(b)
        @pl.when(lax.rem(j, 2) == 0)
        def _():
            it(0)
        @pl.when(lax.rem(j, 2) == 1)
        def _():
            it(1)
        return 0
    lax.fori_loop(0, NCHUNK, _chunk, 0)
    wait_scatter(0)
    wait_scatter(1)

    plsc.subcore_barrier()
    pltpu.sync_copy(agg_sp.at[pl.ds(arow0, _AGG_TPW)],
                    agg_out.at[cid, pl.ds(arow0, _AGG_TPW)])


def _make_edge_pass():
    mesh = plsc.VectorSubcoreMesh(core_axis_name="c", subcore_axis_name="s",
                                  num_cores=NC, num_subcores=NS)
    return pl.kernel(
        _edge_body,
        out_type=jax.ShapeDtypeStruct((NC, NPA, LANES), jnp.float32),
        mesh=mesh,
        scratch_types=[
            pltpu.VMEM((2, 3, CH), jnp.int32),
            pltpu.VMEM((2, CH), jnp.int32),
            pltpu.VMEM((2, CH), jnp.int32),
            pltpu.VMEM((2, CH + LANES), jnp.float32),
            pltpu.VMEM((NKEY,), jnp.float32),
            pltpu.VMEM((2, CH, LANES), jnp.float32),
            pltpu.VMEM((2, CH, LANES), jnp.float32),
            pltpu.SemaphoreType.DMA, pltpu.SemaphoreType.DMA,
            pltpu.SemaphoreType.DMA, pltpu.SemaphoreType.DMA,
            pltpu.SemaphoreType.DMA, pltpu.SemaphoreType.DMA,
            pltpu.VMEM_SHARED((NPA, LANES), jnp.float32),
        ],
        compiler_params=pltpu.CompilerParams(needs_layout_passes=False,
                                             use_tc_tiling_on_sc=False),
    )


_RB = 1000


def _tc1_body(emb, wfc, bfc, wcat, root, bias, z_out, rp_out):
    h = jnp.dot(emb[...], wfc[...], preferred_element_type=jnp.float32) + bfc[...]
    z_out[...] = jnp.dot(h, wcat[...], preferred_element_type=jnp.float32)
    rp_out[...] = jnp.dot(h, root[...], preferred_element_type=jnp.float32) + bias[...]


def _tc1(emb, wfc, bfc, wcat, root, bias):
    f32 = jnp.float32
    full = lambda shape: pl.BlockSpec(shape, lambda i: (0, 0))
    return pl.pallas_call(
        _tc1_body,
        grid=(N // _RB,),
        in_specs=[
            pl.BlockSpec((_RB, D_EMB), lambda i: (i, 0)),
            full((D_EMB, D_IN)), full((1, D_IN)),
            full((D_IN, R * H)), full((D_IN, H)), full((1, H)),
        ],
        out_specs=[pl.BlockSpec((_RB, R * H), lambda i: (i, 0)),
                   pl.BlockSpec((_RB, H), lambda i: (i, 0))],
        out_shape=[jax.ShapeDtypeStruct((N, R * H), f32),
                   jax.ShapeDtypeStruct((N, H), f32)],
    )(emb, wfc, bfc, wcat, root, bias)


def _tc2_body(a0, a1, rp1, wcat2, root2, bias2, z2_out, rp2_out):
    x1 = jnp.maximum(a0[...] + a1[...] + rp1[...], 0.0)
    z2_out[...] = jnp.dot(x1, wcat2[...], preferred_element_type=jnp.float32)
    rp2_out[...] = jnp.dot(x1, root2[...], preferred_element_type=jnp.float32) + bias2[...]


def _tc2(a0, a1, rp1, wcat2, root2, bias2):
    f32 = jnp.float32
    full = lambda shape: pl.BlockSpec(shape, lambda i: (0, 0))
    blk = lambda w: pl.BlockSpec((_RB, w), lambda i: (i, 0))
    return pl.pallas_call(
        _tc2_body,
        grid=(N // _RB,),
        in_specs=[blk(H), blk(H), blk(H),
                  full((H, R * H)), full((H, H)), full((1, H))],
        out_specs=[blk(R * H), blk(H)],
        out_shape=[jax.ShapeDtypeStruct((N, R * H), f32),
                   jax.ShapeDtypeStruct((N, H), f32)],
    )(a0, a1, rp1, wcat2, root2, bias2)


def _tc3_body(a0, a1, rp2, out):
    out[...] = a0[...] + a1[...] + rp2[...]


def _tc3(a0, a1, rp2):
    blk = lambda w: pl.BlockSpec((_RB, w), lambda i: (i, 0))
    return pl.pallas_call(
        _tc3_body,
        grid=(N // _RB,),
        in_specs=[blk(H), blk(H), blk(H)],
        out_specs=blk(H),
        out_shape=jax.ShapeDtypeStruct((N, H), jnp.float32),
    )(a0, a1, rp2)


def kernel(embeddings, edge_index, edge_type, W_fc, b_fc,
           bases1, comp1, root1, bias1, bases2, comp2, root2, bias2):
    f32 = jnp.float32
    src = edge_index[0].astype(jnp.int32)
    dst = edge_index[1].astype(jnp.int32)
    et = edge_type.astype(jnp.int32)
    npad = EP - E
    src_p = jnp.concatenate([src, jnp.zeros((npad,), jnp.int32)])
    dst_p = jnp.concatenate([dst, jnp.full((npad,), N, jnp.int32)])
    et_p = jnp.concatenate([et, jnp.zeros((npad,), jnp.int32)])
    edges3 = jnp.stack([src_p.reshape(TCHUNKS, CH),
                        dst_p.reshape(TCHUNKS, CH),
                        et_p.reshape(TCHUNKS, CH)], axis=1)

    wcat1 = jnp.einsum('rb,bio->rio', comp1, bases1).transpose(1, 0, 2).reshape(D_IN, R * H)
    wcat2 = jnp.einsum('rb,bio->rio', comp2, bases2).transpose(1, 0, 2).reshape(H, R * H)
    zeros_hbm = jnp.zeros((_AGG_TPW, LANES), f32)
    zeros1_hbm = jnp.zeros((_KEY_TPW,), f32)

    recip_flat = _make_deg_pass()(edges3, zeros1_hbm)

    z1, rp1 = _tc1(embeddings, W_fc, b_fc.reshape(1, D_IN), wcat1,
                   root1, bias1.reshape(1, H))

    edge_pass = _make_edge_pass()
    agg1 = edge_pass(edges3, z1.reshape(N * R, LANES), recip_flat, zeros_hbm)
    a10 = agg1[0, :N]
    a11 = agg1[1, :N]

    z2, rp2 = _tc2(a10, a11, rp1, wcat2, root2, bias2.reshape(1, H))

    agg2 = edge_pass(edges3, z2.reshape(N * R, LANES), recip_flat, zeros_hbm)

    return _tc3(agg2[0, :N], agg2[1, :N], rp2)
